# baseline (speedup 1.0000x reference)
import functools

import jax
import jax.numpy as jnp
from jax import lax
from jax.experimental import pallas as pl
from jax.experimental.pallas import tpu as pltpu
from jax.experimental.pallas import tpu_sc as plsc

N = 10000
E = 320000
D = 128
EPS = 1e-7
MIN_NORM = 1e-15
MAX_NORM = 1e6

NC = 2
NS = 16
NW = NC * NS
EP = E // NW
K_CH = 80
NCH = EP // K_CH
SB = 25
NST = NCH // SB
STRIPE = 624
REM = N - NS * STRIPE

B = 1000
GRID = N // B


def _m0(d):
    return (lax.broadcasted_iota(jnp.int32, (1, d), 1) == 0).astype(jnp.float32)


def _cosh(x):
    e = jnp.exp(jnp.clip(x, -15.0, 15.0))
    return 0.5 * (e + 1.0 / e)


def _sinh(x):
    e = jnp.exp(jnp.clip(x, -15.0, 15.0))
    return 0.5 * (e - 1.0 / e)


def _proj(x, K):
    m0 = _m0(x.shape[-1])
    ysq = jnp.sum(x * x * (1.0 - m0), -1, keepdims=True)
    first = jnp.sqrt(jnp.clip(K + ysq, EPS, None))
    return jnp.where(m0 > 0, first, x)


def _expmap0(u, K, sqrtK):
    m0 = _m0(u.shape[-1])
    xs = u * (1.0 - m0)
    xsq = jnp.sum(xs * xs, -1, keepdims=True)
    x_norm = jnp.sqrt(jnp.clip(xsq, MIN_NORM, None))
    theta = x_norm / sqrtK
    first = sqrtK * _cosh(theta)
    rest = sqrtK * _sinh(theta) * xs / x_norm
    return _proj(jnp.where(m0 > 0, first, rest), K)


def _logmap0(x, K, sqrtK):
    m0 = _m0(x.shape[-1])
    ys = x * (1.0 - m0)
    y_norm = jnp.sqrt(jnp.clip(jnp.sum(ys * ys, -1, keepdims=True), MIN_NORM, None))
    x0 = jnp.sum(x * m0, -1, keepdims=True)
    th = jnp.clip(x0 / sqrtK, 1.0 + EPS, None)
    arc = jnp.log(th + jnp.sqrt(jnp.clip(th * th - 1.0, MIN_NORM, None)))
    return sqrtK * arc * ys / y_norm


def _mobius_add_bias(x, u_b, K, sqrtK):
    m0 = _m0(x.shape[-1])
    x0 = jnp.sum(x * m0, -1, keepdims=True)
    ys = x * (1.0 - m0)
    y_norm = jnp.sqrt(jnp.clip(jnp.sum(ys * ys, -1, keepdims=True), MIN_NORM, None))
    y_unit = ys / y_norm
    v = jnp.where(m0 > 0, -y_norm, (sqrtK - x0) * y_unit)
    alpha = jnp.sum(y_unit * u_b, -1, keepdims=True) / sqrtK
    w = u_b - alpha * v
    ux = jnp.sum(ys * w, -1, keepdims=True)
    first = ux / jnp.clip(x0, MIN_NORM, None)
    v2 = jnp.where(m0 > 0, first, w)
    mdot = jnp.sum(v2 * v2, -1, keepdims=True) - 2.0 * first * first
    normu = jnp.clip(jnp.sqrt(jnp.clip(mdot, EPS, None)), None, MAX_NORM)
    theta = jnp.clip(normu / sqrtK, MIN_NORM, None)
    res = _cosh(theta) * x + _sinh(theta) * v2 / theta
    return _proj(res, K)


def _bias_tangent(b_row, K, sqrtK):
    bias1 = b_row * (1.0 - _m0(b_row.shape[-1]))
    return _logmap0(_proj(_expmap0(bias1, K, sqrtK), K), K, sqrtK)


def _tc1_body(c_ref, lin_ref, g1w_ref, linb_ref, g1b_ref, A1_ref, np_ref,
              h1_ref, a1_ref, a2_ref):
    c = c_ref[0, 0]
    K = 1.0 / c
    sqrtK = jnp.sqrt(K)
    m0 = _m0(D)
    A1 = A1_ref[...]
    x_tan = A1 * (1.0 - m0)
    a = _proj(_expmap0(x_tan, K, sqrtK), K)
    ua = _logmap0(a, K, sqrtK)
    ub_lin = _bias_tangent(linb_ref[...], K, sqrtK)
    ub_g1 = _bias_tangent(g1b_ref[...], K, sqrtK)
    mm1 = jnp.dot(ua, lin_ref[...], preferred_element_type=jnp.float32)
    a1 = _mobius_add_bias(_expmap0(mm1, K, sqrtK), ub_lin, K, sqrtK)
    npar = np_ref[...]
    a2 = _expmap0(npar * _logmap0(a1, K, sqrtK), K, sqrtK)
    mmg = jnp.dot(ua, g1w_ref[...], preferred_element_type=jnp.float32)
    h1 = _logmap0(_mobius_add_bias(_expmap0(mmg, K, sqrtK), ub_g1, K, sqrtK),
                  K, sqrtK)
    h1_ref[...] = h1
    a1_ref[...] = a1
    a2_ref[...] = a2


def _tc2_body(c_ref, g2w_ref, g2b_ref, agg_ref, np_ref, a2_ref, h2_ref):
    c = c_ref[0, 0]
    K = 1.0 / c
    sqrtK = jnp.sqrt(K)
    ag = agg_ref[...]
    agg = ag[0] + ag[1]
    x1 = _proj(_expmap0(agg, K, sqrtK), K)
    npar = np_ref[...]
    x1 = _expmap0((1.0 - npar) * _logmap0(x1, K, sqrtK), K, sqrtK)
    x1 = _expmap0(_logmap0(x1, K, sqrtK) + _logmap0(a2_ref[...], K, sqrtK),
                  K, sqrtK)
    ub_g2 = _bias_tangent(g2b_ref[...], K, sqrtK)
    mm = jnp.dot(_logmap0(x1, K, sqrtK), g2w_ref[...],
                 preferred_element_type=jnp.float32)
    h2 = _logmap0(_mobius_add_bias(_expmap0(mm, K, sqrtK), ub_g2, K, sqrtK),
                  K, sqrtK)
    h2_ref[...] = h2


def _tc3_body(c_ref, agg_ref, a1_ref, out_ref):
    c = c_ref[0, 0]
    K = 1.0 / c
    sqrtK = jnp.sqrt(K)
    ag = agg_ref[...]
    agg = ag[0] + ag[1]
    x2 = _proj(_expmap0(agg, K, sqrtK), K)
    l2 = _logmap0(x2, K, sqrtK)
    cat = jnp.concatenate([l2, a1_ref[...]], axis=-1)
    out_ref[...] = _expmap0(cat, K, sqrtK)


_smem_spec = pl.BlockSpec(memory_space=pltpu.SMEM)


def _full_spec(shape):
    nd = len(shape)
    return pl.BlockSpec(shape, lambda i, _n=nd: (0,) * _n)


def _row_spec(d):
    return pl.BlockSpec((B, d), lambda i: (i, 0))


_tc1 = pl.pallas_call(
    _tc1_body,
    grid=(GRID,),
    in_specs=[_smem_spec, _full_spec((D, D)), _full_spec((D, D)),
              _full_spec((1, D)), _full_spec((1, D)),
              _row_spec(D), _row_spec(1)],
    out_specs=[_row_spec(D), _row_spec(D), _row_spec(D)],
    out_shape=[jax.ShapeDtypeStruct((N, D), jnp.float32)] * 3,
)

_tc2 = pl.pallas_call(
    _tc2_body,
    grid=(GRID,),
    in_specs=[_smem_spec, _full_spec((D, D)), _full_spec((1, D)),
              pl.BlockSpec((2, B, D), lambda i: (0, i, 0)),
              _row_spec(1), _row_spec(D)],
    out_specs=[_row_spec(D)],
    out_shape=[jax.ShapeDtypeStruct((N, D), jnp.float32)],
)

_tc3 = pl.pallas_call(
    _tc3_body,
    grid=(GRID,),
    in_specs=[_smem_spec,
              pl.BlockSpec((2, B, D), lambda i: (0, i, 0)),
              _row_spec(D)],
    out_specs=[_row_spec(2 * D)],
    out_shape=[jax.ShapeDtypeStruct((N, 2 * D), jnp.float32)],
)


NBUF = 2


def _sc_agg_body(h_hbm, rows_hbm, cols_hbm, adj_hbm, out_hbm,
                 colv, rowv, adjv, rbuf, aggsh, gsem):
    cid = lax.axis_index("c")
    sid = lax.axis_index("s")
    wid = cid * NS + sid

    def zrow(i, carry):
        for j in range(D // 16):
            rbuf[0, i, pl.ds(16 * j, 16)] = jnp.zeros((16,), jnp.float32)
        return carry

    lax.fori_loop(0, K_CH, zrow, 0)
    for t in range(STRIPE // K_CH):
        pltpu.sync_copy(rbuf.at[0],
                        aggsh.at[pl.ds(sid * STRIPE + t * K_CH, K_CH)])
    pltpu.sync_copy(rbuf.at[0, pl.ds(0, STRIPE % K_CH)],
                    aggsh.at[pl.ds(sid * STRIPE + STRIPE - STRIPE % K_CH,
                                   STRIPE % K_CH)])

    @pl.when(sid == NS - 1)
    def _zero_rem():
        pltpu.sync_copy(rbuf.at[0, pl.ds(0, REM)],
                        aggsh.at[pl.ds(NS * STRIPE, REM)])

    plsc.subcore_barrier()

    def stage(st, carry):
        pltpu.sync_copy(cols_hbm.at[wid, st], colv)
        pltpu.sync_copy(rows_hbm.at[wid, st], rowv)
        pltpu.sync_copy(adj_hbm.at[wid, st], adjv)

        pltpu.async_copy(h_hbm.at[colv.at[0]], rbuf.at[0], gsem.at[0])

        def chunk(g, c1):
            b = lax.rem(g, NBUF)
            bn = 1 - b

            @pl.when(g + 1 < SB)
            def _issue_next():
                pltpu.async_copy(h_hbm.at[colv.at[g + 1]], rbuf.at[bn],
                                 gsem.at[bn])

            pltpu.make_async_copy(h_hbm.at[colv.at[g]], rbuf.at[b],
                                  gsem.at[b]).wait()

            def mul_blk(eb, c2):
                av = adjv[g, pl.ds(16 * eb, 16)]
                for l in range(16):
                    vb = jnp.full((16,), av[l], jnp.float32)
                    e = 16 * eb + l
                    for j in range(D // 16):
                        rbuf[b, e, pl.ds(16 * j, 16)] = (
                            rbuf[b, e, pl.ds(16 * j, 16)] * vb)
                return c2

            lax.fori_loop(0, K_CH // 16, mul_blk, 0)
            pltpu.sync_copy(rbuf.at[b], aggsh.at[rowv.at[g]], add=True)
            return c1

        lax.fori_loop(0, SB, chunk, 0)
        return carry

    lax.fori_loop(0, NST, stage, 0)

    plsc.subcore_barrier()
    pltpu.sync_copy(aggsh.at[pl.ds(sid * STRIPE, STRIPE)],
                    out_hbm.at[cid, pl.ds(sid * STRIPE, STRIPE)])

    @pl.when(sid == NS - 1)
    def _copy_rem():
        pltpu.sync_copy(aggsh.at[pl.ds(NS * STRIPE, REM)],
                        out_hbm.at[cid, pl.ds(NS * STRIPE, REM)])


_sc_agg = functools.partial(
    pl.kernel,
    out_type=jax.ShapeDtypeStruct((NC, N, D), jnp.float32),
    mesh=plsc.VectorSubcoreMesh(core_axis_name="c", subcore_axis_name="s"),
    scratch_types=[
        pltpu.VMEM((SB, K_CH), jnp.int32),
        pltpu.VMEM((SB, K_CH), jnp.int32),
        pltpu.VMEM((SB, K_CH), jnp.float32),
        pltpu.VMEM((NBUF, K_CH, D), jnp.float32),
        pltpu.VMEM_SHARED((N, D), jnp.float32),
        pltpu.SemaphoreType.DMA((NBUF,)),
    ],
)(_sc_agg_body)


def kernel(A1_tensor, adj_values, raw_c, Lin1, Lin1_bias, n_param,
           gc1_w, gc1_b, gc2_w, gc2_b, edge_index):
    c = jax.nn.softplus(raw_c)[0] + 1e-05
    c_arr = jnp.reshape(c, (1, 1))
    linb = Lin1_bias.reshape(1, D)
    g1b = gc1_b.reshape(1, D)
    g2b = gc2_b.reshape(1, D)
    rows2 = edge_index[0].reshape(NW, NST, SB, K_CH)
    cols2 = edge_index[1].reshape(NW, NST, SB, K_CH)
    adj2 = adj_values.reshape(NW, NST, SB, K_CH)

    h1, a1, a2 = _tc1(c_arr, Lin1, gc1_w, linb, g1b, A1_tensor, n_param)
    p1 = _sc_agg(h1, rows2, cols2, adj2)
    (h2,) = _tc2(c_arr, gc2_w, g2b, p1, n_param, a2)
    p2 = _sc_agg(h2, rows2, cols2, adj2)
    (out,) = _tc3(c_arr, p2, a1)
    return out, c


# speedup vs baseline: 2.1508x; 2.1508x over previous
import functools

import jax
import jax.numpy as jnp
from jax import lax
from jax.experimental import pallas as pl
from jax.experimental.pallas import tpu as pltpu
from jax.experimental.pallas import tpu_sc as plsc

N = 10000
E = 320000
D = 128
EPS = 1e-7
MIN_NORM = 1e-15
MAX_NORM = 1e6

NC = 2
NS = 16
NW = NC * NS
EP = E // NW
K_CH = 80
NCH = EP // K_CH
SB = 25
NST = NCH // SB
STRIPE = 624
REM = N - NS * STRIPE

B = 1000
GRID = N // B


def _m0(d):
    return (lax.broadcasted_iota(jnp.int32, (1, d), 1) == 0).astype(jnp.float32)


def _cosh(x):
    e = jnp.exp(jnp.clip(x, -15.0, 15.0))
    return 0.5 * (e + 1.0 / e)


def _sinh(x):
    e = jnp.exp(jnp.clip(x, -15.0, 15.0))
    return 0.5 * (e - 1.0 / e)


def _proj(x, K):
    m0 = _m0(x.shape[-1])
    ysq = jnp.sum(x * x * (1.0 - m0), -1, keepdims=True)
    first = jnp.sqrt(jnp.clip(K + ysq, EPS, None))
    return jnp.where(m0 > 0, first, x)


def _expmap0(u, K, sqrtK):
    m0 = _m0(u.shape[-1])
    xs = u * (1.0 - m0)
    xsq = jnp.sum(xs * xs, -1, keepdims=True)
    x_norm = jnp.sqrt(jnp.clip(xsq, MIN_NORM, None))
    theta = x_norm / sqrtK
    first = sqrtK * _cosh(theta)
    rest = sqrtK * _sinh(theta) * xs / x_norm
    return _proj(jnp.where(m0 > 0, first, rest), K)


def _logmap0(x, K, sqrtK):
    m0 = _m0(x.shape[-1])
    ys = x * (1.0 - m0)
    y_norm = jnp.sqrt(jnp.clip(jnp.sum(ys * ys, -1, keepdims=True), MIN_NORM, None))
    x0 = jnp.sum(x * m0, -1, keepdims=True)
    th = jnp.clip(x0 / sqrtK, 1.0 + EPS, None)
    arc = jnp.log(th + jnp.sqrt(jnp.clip(th * th - 1.0, MIN_NORM, None)))
    return sqrtK * arc * ys / y_norm


def _mobius_add_bias(x, u_b, K, sqrtK):
    m0 = _m0(x.shape[-1])
    x0 = jnp.sum(x * m0, -1, keepdims=True)
    ys = x * (1.0 - m0)
    y_norm = jnp.sqrt(jnp.clip(jnp.sum(ys * ys, -1, keepdims=True), MIN_NORM, None))
    y_unit = ys / y_norm
    v = jnp.where(m0 > 0, -y_norm, (sqrtK - x0) * y_unit)
    alpha = jnp.sum(y_unit * u_b, -1, keepdims=True) / sqrtK
    w = u_b - alpha * v
    ux = jnp.sum(ys * w, -1, keepdims=True)
    first = ux / jnp.clip(x0, MIN_NORM, None)
    v2 = jnp.where(m0 > 0, first, w)
    mdot = jnp.sum(v2 * v2, -1, keepdims=True) - 2.0 * first * first
    normu = jnp.clip(jnp.sqrt(jnp.clip(mdot, EPS, None)), None, MAX_NORM)
    theta = jnp.clip(normu / sqrtK, MIN_NORM, None)
    res = _cosh(theta) * x + _sinh(theta) * v2 / theta
    return _proj(res, K)


def _bias_tangent(b_row, K, sqrtK):
    bias1 = b_row * (1.0 - _m0(b_row.shape[-1]))
    return _logmap0(_proj(_expmap0(bias1, K, sqrtK), K), K, sqrtK)


def _tc1_body(c_ref, lin_ref, g1w_ref, linb_ref, g1b_ref, A1_ref, np_ref,
              h1_ref, a1_ref, a2_ref):
    c = c_ref[0, 0]
    K = 1.0 / c
    sqrtK = jnp.sqrt(K)
    m0 = _m0(D)
    A1 = A1_ref[...]
    x_tan = A1 * (1.0 - m0)
    a = _proj(_expmap0(x_tan, K, sqrtK), K)
    ua = _logmap0(a, K, sqrtK)
    ub_lin = _bias_tangent(linb_ref[...], K, sqrtK)
    ub_g1 = _bias_tangent(g1b_ref[...], K, sqrtK)
    mm1 = jnp.dot(ua, lin_ref[...], preferred_element_type=jnp.float32)
    a1 = _mobius_add_bias(_expmap0(mm1, K, sqrtK), ub_lin, K, sqrtK)
    npar = np_ref[...]
    a2 = _expmap0(npar * _logmap0(a1, K, sqrtK), K, sqrtK)
    mmg = jnp.dot(ua, g1w_ref[...], preferred_element_type=jnp.float32)
    h1 = _logmap0(_mobius_add_bias(_expmap0(mmg, K, sqrtK), ub_g1, K, sqrtK),
                  K, sqrtK)
    h1_ref[...] = h1
    a1_ref[...] = a1
    a2_ref[...] = a2


def _tc2_body(c_ref, g2w_ref, g2b_ref, agg_ref, np_ref, a2_ref, h2_ref):
    c = c_ref[0, 0]
    K = 1.0 / c
    sqrtK = jnp.sqrt(K)
    ag = agg_ref[...]
    agg = ag[0] + ag[1]
    x1 = _proj(_expmap0(agg, K, sqrtK), K)
    npar = np_ref[...]
    x1 = _expmap0((1.0 - npar) * _logmap0(x1, K, sqrtK), K, sqrtK)
    x1 = _expmap0(_logmap0(x1, K, sqrtK) + _logmap0(a2_ref[...], K, sqrtK),
                  K, sqrtK)
    ub_g2 = _bias_tangent(g2b_ref[...], K, sqrtK)
    mm = jnp.dot(_logmap0(x1, K, sqrtK), g2w_ref[...],
                 preferred_element_type=jnp.float32)
    h2 = _logmap0(_mobius_add_bias(_expmap0(mm, K, sqrtK), ub_g2, K, sqrtK),
                  K, sqrtK)
    h2_ref[...] = h2


def _tc3_body(c_ref, agg_ref, a1_ref, out_ref):
    c = c_ref[0, 0]
    K = 1.0 / c
    sqrtK = jnp.sqrt(K)
    ag = agg_ref[...]
    agg = ag[0] + ag[1]
    x2 = _proj(_expmap0(agg, K, sqrtK), K)
    l2 = _logmap0(x2, K, sqrtK)
    cat = jnp.concatenate([l2, a1_ref[...]], axis=-1)
    out_ref[...] = _expmap0(cat, K, sqrtK)


_smem_spec = pl.BlockSpec(memory_space=pltpu.SMEM)


def _full_spec(shape):
    nd = len(shape)
    return pl.BlockSpec(shape, lambda i, _n=nd: (0,) * _n)


def _row_spec(d):
    return pl.BlockSpec((B, d), lambda i: (i, 0))


_tc1 = pl.pallas_call(
    _tc1_body,
    grid=(GRID,),
    in_specs=[_smem_spec, _full_spec((D, D)), _full_spec((D, D)),
              _full_spec((1, D)), _full_spec((1, D)),
              _row_spec(D), _row_spec(1)],
    out_specs=[_row_spec(D), _row_spec(D), _row_spec(D)],
    out_shape=[jax.ShapeDtypeStruct((N, D), jnp.float32)] * 3,
)

_tc2 = pl.pallas_call(
    _tc2_body,
    grid=(GRID,),
    in_specs=[_smem_spec, _full_spec((D, D)), _full_spec((1, D)),
              pl.BlockSpec((2, B, D), lambda i: (0, i, 0)),
              _row_spec(1), _row_spec(D)],
    out_specs=[_row_spec(D)],
    out_shape=[jax.ShapeDtypeStruct((N, D), jnp.float32)],
)

_tc3 = pl.pallas_call(
    _tc3_body,
    grid=(GRID,),
    in_specs=[_smem_spec,
              pl.BlockSpec((2, B, D), lambda i: (0, i, 0)),
              _row_spec(D)],
    out_specs=[_row_spec(2 * D)],
    out_shape=[jax.ShapeDtypeStruct((N, 2 * D), jnp.float32)],
)


def _sc_agg_body(h_hbm, rows_hbm, cols_hbm, adj_hbm, out_hbm,
                 colv, rowv, adjv, rbuf0, rbuf1, aggsh, gsem0, gsem1):
    cid = lax.axis_index("c")
    sid = lax.axis_index("s")
    wid = cid * NS + sid

    def zrow(i, carry):
        for j in range(D // 16):
            rbuf0[i, pl.ds(16 * j, 16)] = jnp.zeros((16,), jnp.float32)
        return carry

    lax.fori_loop(0, K_CH, zrow, 0)
    for t in range(STRIPE // K_CH):
        pltpu.sync_copy(rbuf0,
                        aggsh.at[pl.ds(sid * STRIPE + t * K_CH, K_CH)])
    pltpu.sync_copy(rbuf0.at[pl.ds(0, STRIPE % K_CH)],
                    aggsh.at[pl.ds(sid * STRIPE + STRIPE - STRIPE % K_CH,
                                   STRIPE % K_CH)])

    @pl.when(sid == NS - 1)
    def _zero_rem():
        pltpu.sync_copy(rbuf0.at[pl.ds(0, REM)],
                        aggsh.at[pl.ds(NS * STRIPE, REM)])

    plsc.subcore_barrier()

    def _compute_scatter(g, rb):
        def mul_blk(eb, c2):
            av = adjv[g, pl.ds(16 * eb, 16)]
            for l in range(16):
                vb = jnp.full((16,), av[l], jnp.float32)
                e = 16 * eb + l
                for j in range(D // 16):
                    rb[e, pl.ds(16 * j, 16)] = rb[e, pl.ds(16 * j, 16)] * vb
            return c2

        lax.fori_loop(0, K_CH // 16, mul_blk, 0)
        pltpu.sync_copy(rb, aggsh.at[rowv.at[g]], add=True)

    def stage(st, carry):
        pltpu.sync_copy(cols_hbm.at[wid, st], colv)
        pltpu.sync_copy(rows_hbm.at[wid, st], rowv)
        pltpu.sync_copy(adj_hbm.at[wid, st], adjv)

        pltpu.async_copy(h_hbm.at[colv.at[0]], rbuf0, gsem0)

        def pair(p, c1):
            g0 = 2 * p
            pltpu.async_copy(h_hbm.at[colv.at[g0 + 1]], rbuf1, gsem1)
            pltpu.make_async_copy(h_hbm.at[colv.at[g0]], rbuf0,
                                  gsem0).wait()
            _compute_scatter(g0, rbuf0)
            pltpu.async_copy(h_hbm.at[colv.at[g0 + 2]], rbuf0, gsem0)
            pltpu.make_async_copy(h_hbm.at[colv.at[g0 + 1]], rbuf1,
                                  gsem1).wait()
            _compute_scatter(g0 + 1, rbuf1)
            return c1

        lax.fori_loop(0, (SB - 1) // 2, pair, 0)
        pltpu.make_async_copy(h_hbm.at[colv.at[SB - 1]], rbuf0,
                              gsem0).wait()
        _compute_scatter(SB - 1, rbuf0)
        return carry

    lax.fori_loop(0, NST, stage, 0)

    plsc.subcore_barrier()
    pltpu.sync_copy(aggsh.at[pl.ds(sid * STRIPE, STRIPE)],
                    out_hbm.at[cid, pl.ds(sid * STRIPE, STRIPE)])

    @pl.when(sid == NS - 1)
    def _copy_rem():
        pltpu.sync_copy(aggsh.at[pl.ds(NS * STRIPE, REM)],
                        out_hbm.at[cid, pl.ds(NS * STRIPE, REM)])


_sc_agg = functools.partial(
    pl.kernel,
    out_type=jax.ShapeDtypeStruct((NC, N, D), jnp.float32),
    mesh=plsc.VectorSubcoreMesh(core_axis_name="c", subcore_axis_name="s"),
    scratch_types=[
        pltpu.VMEM((SB, K_CH), jnp.int32),
        pltpu.VMEM((SB, K_CH), jnp.int32),
        pltpu.VMEM((SB, K_CH), jnp.float32),
        pltpu.VMEM((K_CH, D), jnp.float32),
        pltpu.VMEM((K_CH, D), jnp.float32),
        pltpu.VMEM_SHARED((N, D), jnp.float32),
        pltpu.SemaphoreType.DMA,
        pltpu.SemaphoreType.DMA,
    ],
)(_sc_agg_body)


def kernel(A1_tensor, adj_values, raw_c, Lin1, Lin1_bias, n_param,
           gc1_w, gc1_b, gc2_w, gc2_b, edge_index):
    c = jax.nn.softplus(raw_c)[0] + 1e-05
    c_arr = jnp.reshape(c, (1, 1))
    linb = Lin1_bias.reshape(1, D)
    g1b = gc1_b.reshape(1, D)
    g2b = gc2_b.reshape(1, D)
    rows2 = edge_index[0].reshape(NW, NST, SB, K_CH)
    cols2 = edge_index[1].reshape(NW, NST, SB, K_CH)
    adj2 = adj_values.reshape(NW, NST, SB, K_CH)

    h1, a1, a2 = _tc1(c_arr, Lin1, gc1_w, linb, g1b, A1_tensor, n_param)
    p1 = _sc_agg(h1, rows2, cols2, adj2)
    (h2,) = _tc2(c_arr, gc2_w, g2b, p1, n_param, a2)
    p2 = _sc_agg(h2, rows2, cols2, adj2)
    (out,) = _tc3(c_arr, p2, a1)
    return out, c


# speedup vs baseline: 2.4307x; 1.1301x over previous
import functools

import jax
import jax.numpy as jnp
from jax import lax
from jax.experimental import pallas as pl
from jax.experimental.pallas import tpu as pltpu
from jax.experimental.pallas import tpu_sc as plsc

N = 10000
E = 320000
D = 128
EPS = 1e-7
MIN_NORM = 1e-15
MAX_NORM = 1e6

NC = 2
NS = 16
NW = NC * NS
EP = E // NW
K_CH = 80
NCH = EP // K_CH
SB = 25
NST = NCH // SB
STRIPE = 624
REM = N - NS * STRIPE

B = 1000
GRID = N // B


def _m0(d):
    return (lax.broadcasted_iota(jnp.int32, (1, d), 1) == 0).astype(jnp.float32)


def _cosh(x):
    e = jnp.exp(jnp.clip(x, -15.0, 15.0))
    return 0.5 * (e + 1.0 / e)


def _sinh(x):
    e = jnp.exp(jnp.clip(x, -15.0, 15.0))
    return 0.5 * (e - 1.0 / e)


def _proj(x, K):
    m0 = _m0(x.shape[-1])
    ysq = jnp.sum(x * x * (1.0 - m0), -1, keepdims=True)
    first = jnp.sqrt(jnp.clip(K + ysq, EPS, None))
    return jnp.where(m0 > 0, first, x)


def _expmap0(u, K, sqrtK):
    m0 = _m0(u.shape[-1])
    xs = u * (1.0 - m0)
    xsq = jnp.sum(xs * xs, -1, keepdims=True)
    x_norm = jnp.sqrt(jnp.clip(xsq, MIN_NORM, None))
    theta = x_norm / sqrtK
    first = sqrtK * _cosh(theta)
    rest = sqrtK * _sinh(theta) * xs / x_norm
    return _proj(jnp.where(m0 > 0, first, rest), K)


def _logmap0(x, K, sqrtK):
    m0 = _m0(x.shape[-1])
    ys = x * (1.0 - m0)
    y_norm = jnp.sqrt(jnp.clip(jnp.sum(ys * ys, -1, keepdims=True), MIN_NORM, None))
    x0 = jnp.sum(x * m0, -1, keepdims=True)
    th = jnp.clip(x0 / sqrtK, 1.0 + EPS, None)
    arc = jnp.log(th + jnp.sqrt(jnp.clip(th * th - 1.0, MIN_NORM, None)))
    return sqrtK * arc * ys / y_norm


def _mobius_add_bias(x, u_b, K, sqrtK):
    m0 = _m0(x.shape[-1])
    x0 = jnp.sum(x * m0, -1, keepdims=True)
    ys = x * (1.0 - m0)
    y_norm = jnp.sqrt(jnp.clip(jnp.sum(ys * ys, -1, keepdims=True), MIN_NORM, None))
    y_unit = ys / y_norm
    v = jnp.where(m0 > 0, -y_norm, (sqrtK - x0) * y_unit)
    alpha = jnp.sum(y_unit * u_b, -1, keepdims=True) / sqrtK
    w = u_b - alpha * v
    ux = jnp.sum(ys * w, -1, keepdims=True)
    first = ux / jnp.clip(x0, MIN_NORM, None)
    v2 = jnp.where(m0 > 0, first, w)
    mdot = jnp.sum(v2 * v2, -1, keepdims=True) - 2.0 * first * first
    normu = jnp.clip(jnp.sqrt(jnp.clip(mdot, EPS, None)), None, MAX_NORM)
    theta = jnp.clip(normu / sqrtK, MIN_NORM, None)
    res = _cosh(theta) * x + _sinh(theta) * v2 / theta
    return _proj(res, K)


def _bias_tangent(b_row, K, sqrtK):
    bias1 = b_row * (1.0 - _m0(b_row.shape[-1]))
    return _logmap0(_proj(_expmap0(bias1, K, sqrtK), K), K, sqrtK)


def _tclip(u, sqrtK):
    m0 = _m0(u.shape[-1])
    us = u * (1.0 - m0)
    n = jnp.sqrt(jnp.clip(jnp.sum(us * us, -1, keepdims=True), MIN_NORM, None))
    return us * jnp.minimum(1.0, 15.0 * sqrtK / n)


def _tc1_body(c_ref, lin_ref, g1w_ref, linb_ref, g1b_ref, A1_ref, np_ref,
              h1_ref, a1_ref, a2_ref):
    c = c_ref[0, 0]
    K = 1.0 / c
    sqrtK = jnp.sqrt(K)
    ua = _tclip(A1_ref[...], sqrtK)
    ub_lin = _bias_tangent(linb_ref[...], K, sqrtK)
    ub_g1 = _bias_tangent(g1b_ref[...], K, sqrtK)
    mm1 = jnp.dot(ua, lin_ref[...], preferred_element_type=jnp.float32)
    a1 = _mobius_add_bias(_expmap0(mm1, K, sqrtK), ub_lin, K, sqrtK)
    npar = np_ref[...]
    t_a2 = _tclip(npar * _logmap0(a1, K, sqrtK), sqrtK)
    mmg = jnp.dot(ua, g1w_ref[...], preferred_element_type=jnp.float32)
    h1 = _logmap0(_mobius_add_bias(_expmap0(mmg, K, sqrtK), ub_g1, K, sqrtK),
                  K, sqrtK)
    h1_ref[...] = h1
    a1_ref[...] = a1
    a2_ref[...] = t_a2


def _tc2_body(c_ref, g2w_ref, g2b_ref, agg_ref, np_ref, a2_ref, h2_ref):
    c = c_ref[0, 0]
    K = 1.0 / c
    sqrtK = jnp.sqrt(K)
    ag = agg_ref[...]
    agg = ag[0] + ag[1]
    npar = np_ref[...]
    u2 = _tclip(_tclip((1.0 - npar) * _tclip(agg, sqrtK), sqrtK)
                + a2_ref[...], sqrtK)
    ub_g2 = _bias_tangent(g2b_ref[...], K, sqrtK)
    mm = jnp.dot(u2, g2w_ref[...], preferred_element_type=jnp.float32)
    h2 = _logmap0(_mobius_add_bias(_expmap0(mm, K, sqrtK), ub_g2, K, sqrtK),
                  K, sqrtK)
    h2_ref[...] = h2


def _tc3_body(c_ref, agg_ref, a1_ref, out_ref):
    c = c_ref[0, 0]
    K = 1.0 / c
    sqrtK = jnp.sqrt(K)
    ag = agg_ref[...]
    agg = ag[0] + ag[1]
    cat = jnp.concatenate([_tclip(agg, sqrtK), a1_ref[...]], axis=-1)
    out_ref[...] = _expmap0(cat, K, sqrtK)


_smem_spec = pl.BlockSpec(memory_space=pltpu.SMEM)


def _full_spec(shape):
    nd = len(shape)
    return pl.BlockSpec(shape, lambda i, _n=nd: (0,) * _n)


def _row_spec(d):
    return pl.BlockSpec((B, d), lambda i: (i, 0))


_tc1 = pl.pallas_call(
    _tc1_body,
    grid=(GRID,),
    in_specs=[_smem_spec, _full_spec((D, D)), _full_spec((D, D)),
              _full_spec((1, D)), _full_spec((1, D)),
              _row_spec(D), _row_spec(1)],
    out_specs=[_row_spec(D), _row_spec(D), _row_spec(D)],
    out_shape=[jax.ShapeDtypeStruct((N, D), jnp.float32)] * 3,
)

_tc2 = pl.pallas_call(
    _tc2_body,
    grid=(GRID,),
    in_specs=[_smem_spec, _full_spec((D, D)), _full_spec((1, D)),
              pl.BlockSpec((2, B, D), lambda i: (0, i, 0)),
              _row_spec(1), _row_spec(D)],
    out_specs=[_row_spec(D)],
    out_shape=[jax.ShapeDtypeStruct((N, D), jnp.float32)],
)

_tc3 = pl.pallas_call(
    _tc3_body,
    grid=(GRID,),
    in_specs=[_smem_spec,
              pl.BlockSpec((2, B, D), lambda i: (0, i, 0)),
              _row_spec(D)],
    out_specs=[_row_spec(2 * D)],
    out_shape=[jax.ShapeDtypeStruct((N, 2 * D), jnp.float32)],
)


def _sc_agg_body(h_hbm, rows_hbm, cols_hbm, adj_hbm, out_hbm,
                 colv, rowv, adjv, rbuf0, rbuf1, aggsh, gsem0, gsem1):
    cid = lax.axis_index("c")
    sid = lax.axis_index("s")
    wid = cid * NS + sid

    def zrow(i, carry):
        for j in range(D // 16):
            rbuf0[i, pl.ds(16 * j, 16)] = jnp.zeros((16,), jnp.float32)
        return carry

    lax.fori_loop(0, K_CH, zrow, 0)
    for t in range(STRIPE // K_CH):
        pltpu.sync_copy(rbuf0,
                        aggsh.at[pl.ds(sid * STRIPE + t * K_CH, K_CH)])
    pltpu.sync_copy(rbuf0.at[pl.ds(0, STRIPE % K_CH)],
                    aggsh.at[pl.ds(sid * STRIPE + STRIPE - STRIPE % K_CH,
                                   STRIPE % K_CH)])

    @pl.when(sid == NS - 1)
    def _zero_rem():
        pltpu.sync_copy(rbuf0.at[pl.ds(0, REM)],
                        aggsh.at[pl.ds(NS * STRIPE, REM)])

    plsc.subcore_barrier()

    def _compute_scatter(g, rb):
        def mul_blk(eb, c2):
            av = adjv[g, pl.ds(16 * eb, 16)]
            for l in range(16):
                vb = jnp.full((16,), av[l], jnp.float32)
                e = 16 * eb + l
                for j in range(D // 16):
                    rb[e, pl.ds(16 * j, 16)] = rb[e, pl.ds(16 * j, 16)] * vb
            return c2

        lax.fori_loop(0, K_CH // 16, mul_blk, 0)
        pltpu.sync_copy(rb, aggsh.at[rowv.at[g]], add=True)

    def stage(st, carry):
        pltpu.sync_copy(cols_hbm.at[wid, st], colv)
        pltpu.sync_copy(rows_hbm.at[wid, st], rowv)
        pltpu.sync_copy(adj_hbm.at[wid, st], adjv)

        pltpu.async_copy(h_hbm.at[colv.at[0]], rbuf0, gsem0)

        def pair(p, c1):
            g0 = 2 * p
            pltpu.async_copy(h_hbm.at[colv.at[g0 + 1]], rbuf1, gsem1)
            pltpu.make_async_copy(h_hbm.at[colv.at[g0]], rbuf0,
                                  gsem0).wait()
            _compute_scatter(g0, rbuf0)
            pltpu.async_copy(h_hbm.at[colv.at[g0 + 2]], rbuf0, gsem0)
            pltpu.make_async_copy(h_hbm.at[colv.at[g0 + 1]], rbuf1,
                                  gsem1).wait()
            _compute_scatter(g0 + 1, rbuf1)
            return c1

        lax.fori_loop(0, (SB - 1) // 2, pair, 0)
        pltpu.make_async_copy(h_hbm.at[colv.at[SB - 1]], rbuf0,
                              gsem0).wait()
        _compute_scatter(SB - 1, rbuf0)
        return carry

    lax.fori_loop(0, NST, stage, 0)

    plsc.subcore_barrier()
    pltpu.sync_copy(aggsh.at[pl.ds(sid * STRIPE, STRIPE)],
                    out_hbm.at[cid, pl.ds(sid * STRIPE, STRIPE)])

    @pl.when(sid == NS - 1)
    def _copy_rem():
        pltpu.sync_copy(aggsh.at[pl.ds(NS * STRIPE, REM)],
                        out_hbm.at[cid, pl.ds(NS * STRIPE, REM)])


_sc_agg = functools.partial(
    pl.kernel,
    out_type=jax.ShapeDtypeStruct((NC, N, D), jnp.float32),
    mesh=plsc.VectorSubcoreMesh(core_axis_name="c", subcore_axis_name="s"),
    scratch_types=[
        pltpu.VMEM((SB, K_CH), jnp.int32),
        pltpu.VMEM((SB, K_CH), jnp.int32),
        pltpu.VMEM((SB, K_CH), jnp.float32),
        pltpu.VMEM((K_CH, D), jnp.float32),
        pltpu.VMEM((K_CH, D), jnp.float32),
        pltpu.VMEM_SHARED((N, D), jnp.float32),
        pltpu.SemaphoreType.DMA,
        pltpu.SemaphoreType.DMA,
    ],
)(_sc_agg_body)


def kernel(A1_tensor, adj_values, raw_c, Lin1, Lin1_bias, n_param,
           gc1_w, gc1_b, gc2_w, gc2_b, edge_index):
    c = jax.nn.softplus(raw_c)[0] + 1e-05
    c_arr = jnp.reshape(c, (1, 1))
    linb = Lin1_bias.reshape(1, D)
    g1b = gc1_b.reshape(1, D)
    g2b = gc2_b.reshape(1, D)
    rows2 = edge_index[0].reshape(NW, NST, SB, K_CH)
    cols2 = edge_index[1].reshape(NW, NST, SB, K_CH)
    adj2 = adj_values.reshape(NW, NST, SB, K_CH)

    h1, a1, a2 = _tc1(c_arr, Lin1, gc1_w, linb, g1b, A1_tensor, n_param)
    p1 = _sc_agg(h1, rows2, cols2, adj2)
    (h2,) = _tc2(c_arr, gc2_w, g2b, p1, n_param, a2)
    p2 = _sc_agg(h2, rows2, cols2, adj2)
    (out,) = _tc3(c_arr, p2, a1)
    return out, c


# speedup vs baseline: 2.6541x; 1.0919x over previous
import functools

import jax
import jax.numpy as jnp
from jax import lax
from jax.experimental import pallas as pl
from jax.experimental.pallas import tpu as pltpu
from jax.experimental.pallas import tpu_sc as plsc

N = 10000
E = 320000
D = 128
EPS = 1e-7
MIN_NORM = 1e-15
MAX_NORM = 1e6

NC = 2
NS = 16
NW = NC * NS
EP = E // NW
K_CH = 80
NCH = EP // K_CH
SB = 25
NST = NCH // SB
STRIPE = 624
REM = N - NS * STRIPE

B = 1000
GRID = N // B


def _m0(d):
    return (lax.broadcasted_iota(jnp.int32, (1, d), 1) == 0).astype(jnp.float32)


def _cosh(x):
    e = jnp.exp(jnp.clip(x, -15.0, 15.0))
    return 0.5 * (e + 1.0 / e)


def _sinh(x):
    e = jnp.exp(jnp.clip(x, -15.0, 15.0))
    return 0.5 * (e - 1.0 / e)


def _proj(x, K):
    m0 = _m0(x.shape[-1])
    ysq = jnp.sum(x * x * (1.0 - m0), -1, keepdims=True)
    first = jnp.sqrt(jnp.clip(K + ysq, EPS, None))
    return jnp.where(m0 > 0, first, x)


def _expmap0(u, K, sqrtK):
    m0 = _m0(u.shape[-1])
    xs = u * (1.0 - m0)
    xsq = jnp.sum(xs * xs, -1, keepdims=True)
    x_norm = jnp.sqrt(jnp.clip(xsq, MIN_NORM, None))
    theta = x_norm / sqrtK
    first = sqrtK * _cosh(theta)
    rest = sqrtK * _sinh(theta) * xs / x_norm
    return _proj(jnp.where(m0 > 0, first, rest), K)


def _logmap0(x, K, sqrtK):
    m0 = _m0(x.shape[-1])
    ys = x * (1.0 - m0)
    y_norm = jnp.sqrt(jnp.clip(jnp.sum(ys * ys, -1, keepdims=True), MIN_NORM, None))
    x0 = jnp.sum(x * m0, -1, keepdims=True)
    th = jnp.clip(x0 / sqrtK, 1.0 + EPS, None)
    arc = jnp.log(th + jnp.sqrt(jnp.clip(th * th - 1.0, MIN_NORM, None)))
    return sqrtK * arc * ys / y_norm


def _mobius_add_bias(x, u_b, K, sqrtK):
    m0 = _m0(x.shape[-1])
    x0 = jnp.sum(x * m0, -1, keepdims=True)
    ys = x * (1.0 - m0)
    y_norm = jnp.sqrt(jnp.clip(jnp.sum(ys * ys, -1, keepdims=True), MIN_NORM, None))
    y_unit = ys / y_norm
    v = jnp.where(m0 > 0, -y_norm, (sqrtK - x0) * y_unit)
    alpha = jnp.sum(y_unit * u_b, -1, keepdims=True) / sqrtK
    w = u_b - alpha * v
    ux = jnp.sum(ys * w, -1, keepdims=True)
    first = ux / jnp.clip(x0, MIN_NORM, None)
    v2 = jnp.where(m0 > 0, first, w)
    mdot = jnp.sum(v2 * v2, -1, keepdims=True) - 2.0 * first * first
    normu = jnp.clip(jnp.sqrt(jnp.clip(mdot, EPS, None)), None, MAX_NORM)
    theta = jnp.clip(normu / sqrtK, MIN_NORM, None)
    res = _cosh(theta) * x + _sinh(theta) * v2 / theta
    return _proj(res, K)


def _bias_tangent(b_row, K, sqrtK):
    bias1 = b_row * (1.0 - _m0(b_row.shape[-1]))
    return _logmap0(_proj(_expmap0(bias1, K, sqrtK), K), K, sqrtK)


def _tclip(u, sqrtK):
    m0 = _m0(u.shape[-1])
    us = u * (1.0 - m0)
    n = jnp.sqrt(jnp.clip(jnp.sum(us * us, -1, keepdims=True), MIN_NORM, None))
    return us * jnp.minimum(1.0, 15.0 * sqrtK / n)


def _tc1_body(c_ref, lin_ref, g1w_ref, linb_ref, g1b_ref, A1_ref, np_ref,
              h1_ref, a1_ref, a2_ref):
    c = c_ref[0, 0]
    K = 1.0 / c
    sqrtK = jnp.sqrt(K)
    ua = _tclip(A1_ref[...], sqrtK)
    ub_lin = _bias_tangent(linb_ref[...], K, sqrtK)
    ub_g1 = _bias_tangent(g1b_ref[...], K, sqrtK)
    mm1 = jnp.dot(ua, lin_ref[...], preferred_element_type=jnp.float32)
    a1 = _mobius_add_bias(_expmap0(mm1, K, sqrtK), ub_lin, K, sqrtK)
    npar = np_ref[...]
    t_a2 = _tclip(npar * _logmap0(a1, K, sqrtK), sqrtK)
    mmg = jnp.dot(ua, g1w_ref[...], preferred_element_type=jnp.float32)
    h1 = _logmap0(_mobius_add_bias(_expmap0(mmg, K, sqrtK), ub_g1, K, sqrtK),
                  K, sqrtK)
    h1_ref[...] = h1
    a1_ref[...] = a1
    a2_ref[...] = t_a2


def _tc2_body(c_ref, g2w_ref, g2b_ref, agg_ref, np_ref, a2_ref, h2_ref):
    c = c_ref[0, 0]
    K = 1.0 / c
    sqrtK = jnp.sqrt(K)
    ag = agg_ref[...]
    agg = ag[0] + ag[1]
    npar = np_ref[...]
    u2 = _tclip(_tclip((1.0 - npar) * _tclip(agg, sqrtK), sqrtK)
                + a2_ref[...], sqrtK)
    ub_g2 = _bias_tangent(g2b_ref[...], K, sqrtK)
    mm = jnp.dot(u2, g2w_ref[...], preferred_element_type=jnp.float32)
    h2 = _logmap0(_mobius_add_bias(_expmap0(mm, K, sqrtK), ub_g2, K, sqrtK),
                  K, sqrtK)
    h2_ref[...] = h2


def _tc3_body(c_ref, agg_ref, a1_ref, out_ref):
    c = c_ref[0, 0]
    K = 1.0 / c
    sqrtK = jnp.sqrt(K)
    ag = agg_ref[...]
    agg = ag[0] + ag[1]
    cat = jnp.concatenate([_tclip(agg, sqrtK), a1_ref[...]], axis=-1)
    out_ref[...] = _expmap0(cat, K, sqrtK)


_smem_spec = pl.BlockSpec(memory_space=pltpu.SMEM)


def _full_spec(shape):
    nd = len(shape)
    return pl.BlockSpec(shape, lambda i, _n=nd: (0,) * _n)


def _row_spec(d):
    return pl.BlockSpec((B, d), lambda i: (i, 0))


_tc1 = pl.pallas_call(
    _tc1_body,
    grid=(GRID,),
    in_specs=[_smem_spec, _full_spec((D, D)), _full_spec((D, D)),
              _full_spec((1, D)), _full_spec((1, D)),
              _row_spec(D), _row_spec(1)],
    out_specs=[_row_spec(D), _row_spec(D), _row_spec(D)],
    out_shape=[jax.ShapeDtypeStruct((N, D), jnp.float32)] * 3,
)

_tc2 = pl.pallas_call(
    _tc2_body,
    grid=(GRID,),
    in_specs=[_smem_spec, _full_spec((D, D)), _full_spec((1, D)),
              pl.BlockSpec((2, B, D), lambda i: (0, i, 0)),
              _row_spec(1), _row_spec(D)],
    out_specs=[_row_spec(D)],
    out_shape=[jax.ShapeDtypeStruct((N, D), jnp.float32)],
)

_tc3 = pl.pallas_call(
    _tc3_body,
    grid=(GRID,),
    in_specs=[_smem_spec,
              pl.BlockSpec((2, B, D), lambda i: (0, i, 0)),
              _row_spec(D)],
    out_specs=[_row_spec(2 * D)],
    out_shape=[jax.ShapeDtypeStruct((N, 2 * D), jnp.float32)],
)


def _sc_agg_body(h_hbm, rows_hbm, cols_hbm, adj_hbm, out_hbm,
                 colv, rowv, adjv, rbuf0, rbuf1, rbuf2, aggsh,
                 gsem0, gsem1, gsem2, ssem0, ssem1, ssem2):
    cid = lax.axis_index("c")
    sid = lax.axis_index("s")
    wid = cid * NS + sid

    def zrow(i, carry):
        for j in range(D // 16):
            rbuf0[i, pl.ds(16 * j, 16)] = jnp.zeros((16,), jnp.float32)
        return carry

    lax.fori_loop(0, K_CH, zrow, 0)
    for t in range(STRIPE // K_CH):
        pltpu.sync_copy(rbuf0,
                        aggsh.at[pl.ds(sid * STRIPE + t * K_CH, K_CH)])
    pltpu.sync_copy(rbuf0.at[pl.ds(0, STRIPE % K_CH)],
                    aggsh.at[pl.ds(sid * STRIPE + STRIPE - STRIPE % K_CH,
                                   STRIPE % K_CH)])

    @pl.when(sid == NS - 1)
    def _zero_rem():
        pltpu.sync_copy(rbuf0.at[pl.ds(0, REM)],
                        aggsh.at[pl.ds(NS * STRIPE, REM)])

    plsc.subcore_barrier()

    def _mul(g, rb):
        def mul_blk(eb, c2):
            av = adjv[g, pl.ds(16 * eb, 16)]
            for l in range(16):
                vb = jnp.full((16,), av[l], jnp.float32)
                e = 16 * eb + l
                for j in range(D // 16):
                    rb[e, pl.ds(16 * j, 16)] = rb[e, pl.ds(16 * j, 16)] * vb
            return c2

        lax.fori_loop(0, K_CH // 16, mul_blk, 0)

    def _gissue(g, rb, sem):
        pltpu.async_copy(h_hbm.at[colv.at[g]], rb, sem)

    def _gwait(g, rb, sem):
        pltpu.make_async_copy(h_hbm.at[colv.at[g]], rb, sem).wait()

    def _sissue(g, rb, sem):
        pltpu.async_copy(rb, aggsh.at[rowv.at[g]], sem, add=True)

    def _swait(rb, sem):
        pltpu.make_async_copy(rb, aggsh.at[rowv.at[0]], sem).wait()

    def stage(st, carry):
        pltpu.sync_copy(cols_hbm.at[wid, st], colv)
        pltpu.sync_copy(rows_hbm.at[wid, st], rowv)
        pltpu.sync_copy(adj_hbm.at[wid, st], adjv)

        pltpu.async_copy(h_hbm.at[colv.at[0]], rbuf0, gsem0)
        pltpu.async_copy(h_hbm.at[colv.at[1]], rbuf1, gsem1)

        _gwait(0, rbuf0, gsem0)
        _mul(0, rbuf0)
        _gissue(2, rbuf2, gsem2)
        _sissue(0, rbuf0, ssem0)

        _gwait(1, rbuf1, gsem1)
        _mul(1, rbuf1)
        _swait(rbuf0, ssem0)
        _gissue(3, rbuf0, gsem0)
        _sissue(1, rbuf1, ssem1)

        def core(p, c1):
            k = 3 * p + 2
            _gwait(k, rbuf2, gsem2)
            _mul(k, rbuf2)
            _swait(rbuf1, ssem1)
            _gissue(k + 2, rbuf1, gsem1)
            _sissue(k, rbuf2, ssem2)

            _gwait(k + 1, rbuf0, gsem0)
            _mul(k + 1, rbuf0)
            _swait(rbuf2, ssem2)
            _gissue(k + 3, rbuf2, gsem2)
            _sissue(k + 1, rbuf0, ssem0)

            _gwait(k + 2, rbuf1, gsem1)
            _mul(k + 2, rbuf1)
            _swait(rbuf0, ssem0)
            _gissue(k + 4, rbuf0, gsem0)
            _sissue(k + 2, rbuf1, ssem1)
            return c1

        lax.fori_loop(0, (SB - 4) // 3, core, 0)

        _gwait(SB - 2, rbuf2, gsem2)
        _mul(SB - 2, rbuf2)
        _swait(rbuf1, ssem1)
        _sissue(SB - 2, rbuf2, ssem2)

        _gwait(SB - 1, rbuf0, gsem0)
        _mul(SB - 1, rbuf0)
        _swait(rbuf2, ssem2)
        _sissue(SB - 1, rbuf0, ssem0)

        _swait(rbuf0, ssem0)
        return carry

    lax.fori_loop(0, NST, stage, 0)

    plsc.subcore_barrier()
    pltpu.sync_copy(aggsh.at[pl.ds(sid * STRIPE, STRIPE)],
                    out_hbm.at[cid, pl.ds(sid * STRIPE, STRIPE)])

    @pl.when(sid == NS - 1)
    def _copy_rem():
        pltpu.sync_copy(aggsh.at[pl.ds(NS * STRIPE, REM)],
                        out_hbm.at[cid, pl.ds(NS * STRIPE, REM)])


_sc_agg = functools.partial(
    pl.kernel,
    out_type=jax.ShapeDtypeStruct((NC, N, D), jnp.float32),
    mesh=plsc.VectorSubcoreMesh(core_axis_name="c", subcore_axis_name="s"),
    scratch_types=[
        pltpu.VMEM((SB, K_CH), jnp.int32),
        pltpu.VMEM((SB, K_CH), jnp.int32),
        pltpu.VMEM((SB, K_CH), jnp.float32),
        pltpu.VMEM((K_CH, D), jnp.float32),
        pltpu.VMEM((K_CH, D), jnp.float32),
        pltpu.VMEM((K_CH, D), jnp.float32),
        pltpu.VMEM_SHARED((N, D), jnp.float32),
        pltpu.SemaphoreType.DMA,
        pltpu.SemaphoreType.DMA,
        pltpu.SemaphoreType.DMA,
        pltpu.SemaphoreType.DMA,
        pltpu.SemaphoreType.DMA,
        pltpu.SemaphoreType.DMA,
    ],
)(_sc_agg_body)


def kernel(A1_tensor, adj_values, raw_c, Lin1, Lin1_bias, n_param,
           gc1_w, gc1_b, gc2_w, gc2_b, edge_index):
    c = jax.nn.softplus(raw_c)[0] + 1e-05
    c_arr = jnp.reshape(c, (1, 1))
    linb = Lin1_bias.reshape(1, D)
    g1b = gc1_b.reshape(1, D)
    g2b = gc2_b.reshape(1, D)
    rows2 = edge_index[0].reshape(NW, NST, SB, K_CH)
    cols2 = edge_index[1].reshape(NW, NST, SB, K_CH)
    adj2 = adj_values.reshape(NW, NST, SB, K_CH)

    h1, a1, a2 = _tc1(c_arr, Lin1, gc1_w, linb, g1b, A1_tensor, n_param)
    p1 = _sc_agg(h1, rows2, cols2, adj2)
    (h2,) = _tc2(c_arr, gc2_w, g2b, p1, n_param, a2)
    p2 = _sc_agg(h2, rows2, cols2, adj2)
    (out,) = _tc3(c_arr, p2, a1)
    return out, c


# speedup vs baseline: 2.8368x; 1.0688x over previous
import functools

import jax
import jax.numpy as jnp
from jax import lax
from jax.experimental import pallas as pl
from jax.experimental.pallas import tpu as pltpu
from jax.experimental.pallas import tpu_sc as plsc

N = 10000
E = 320000
D = 128
EPS = 1e-7
MIN_NORM = 1e-15
MAX_NORM = 1e6

NC = 2
NS = 16
NW = NC * NS
EP = E // NW
K_CH = 80
NCH = EP // K_CH
SB = 25
NST = NCH // SB
STRIPE = 624
REM = N - NS * STRIPE

B = 1000
GRID = N // B


def _m0(d):
    return (lax.broadcasted_iota(jnp.int32, (1, d), 1) == 0).astype(jnp.float32)


def _cosh(x):
    e = jnp.exp(jnp.clip(x, -15.0, 15.0))
    return 0.5 * (e + 1.0 / e)


def _sinh(x):
    e = jnp.exp(jnp.clip(x, -15.0, 15.0))
    return 0.5 * (e - 1.0 / e)


def _proj(x, K):
    m0 = _m0(x.shape[-1])
    ysq = jnp.sum(x * x * (1.0 - m0), -1, keepdims=True)
    first = jnp.sqrt(jnp.clip(K + ysq, EPS, None))
    return jnp.where(m0 > 0, first, x)


def _expmap0(u, K, sqrtK):
    m0 = _m0(u.shape[-1])
    xs = u * (1.0 - m0)
    xsq = jnp.sum(xs * xs, -1, keepdims=True)
    x_norm = jnp.sqrt(jnp.clip(xsq, MIN_NORM, None))
    theta = x_norm / sqrtK
    first = sqrtK * _cosh(theta)
    rest = sqrtK * _sinh(theta) * xs / x_norm
    return _proj(jnp.where(m0 > 0, first, rest), K)


def _logmap0(x, K, sqrtK):
    m0 = _m0(x.shape[-1])
    ys = x * (1.0 - m0)
    y_norm = jnp.sqrt(jnp.clip(jnp.sum(ys * ys, -1, keepdims=True), MIN_NORM, None))
    x0 = jnp.sum(x * m0, -1, keepdims=True)
    th = jnp.clip(x0 / sqrtK, 1.0 + EPS, None)
    arc = jnp.log(th + jnp.sqrt(jnp.clip(th * th - 1.0, MIN_NORM, None)))
    return sqrtK * arc * ys / y_norm


def _mobius_add_bias(x, u_b, K, sqrtK):
    m0 = _m0(x.shape[-1])
    x0 = jnp.sum(x * m0, -1, keepdims=True)
    ys = x * (1.0 - m0)
    y_norm = jnp.sqrt(jnp.clip(jnp.sum(ys * ys, -1, keepdims=True), MIN_NORM, None))
    y_unit = ys / y_norm
    v = jnp.where(m0 > 0, -y_norm, (sqrtK - x0) * y_unit)
    alpha = jnp.sum(y_unit * u_b, -1, keepdims=True) / sqrtK
    w = u_b - alpha * v
    ux = jnp.sum(ys * w, -1, keepdims=True)
    first = ux / jnp.clip(x0, MIN_NORM, None)
    v2 = jnp.where(m0 > 0, first, w)
    mdot = jnp.sum(v2 * v2, -1, keepdims=True) - 2.0 * first * first
    normu = jnp.clip(jnp.sqrt(jnp.clip(mdot, EPS, None)), None, MAX_NORM)
    theta = jnp.clip(normu / sqrtK, MIN_NORM, None)
    res = _cosh(theta) * x + _sinh(theta) * v2 / theta
    return _proj(res, K)


def _bias_tangent(b_row, K, sqrtK):
    bias1 = b_row * (1.0 - _m0(b_row.shape[-1]))
    return _logmap0(_proj(_expmap0(bias1, K, sqrtK), K), K, sqrtK)


def _tclip(u, sqrtK):
    m0 = _m0(u.shape[-1])
    us = u * (1.0 - m0)
    n = jnp.sqrt(jnp.clip(jnp.sum(us * us, -1, keepdims=True), MIN_NORM, None))
    return us * jnp.minimum(1.0, 15.0 * sqrtK / n)


def _tc1h_body(c_ref, g1w_ref, g1b_ref, A1_ref, h1_ref):
    c = c_ref[0, 0]
    K = 1.0 / c
    sqrtK = jnp.sqrt(K)
    ua = _tclip(A1_ref[...], sqrtK)
    ub_g1 = _bias_tangent(g1b_ref[...], K, sqrtK)
    mmg = jnp.dot(ua, g1w_ref[...], preferred_element_type=jnp.float32)
    h1 = _logmap0(_mobius_add_bias(_expmap0(mmg, K, sqrtK), ub_g1, K, sqrtK),
                  K, sqrtK)
    h1_ref[...] = h1


def _tc1a_body(c_ref, lin_ref, linb_ref, A1_ref, np_ref, a1_ref, a2_ref):
    c = c_ref[0, 0]
    K = 1.0 / c
    sqrtK = jnp.sqrt(K)
    ua = _tclip(A1_ref[...], sqrtK)
    ub_lin = _bias_tangent(linb_ref[...], K, sqrtK)
    mm1 = jnp.dot(ua, lin_ref[...], preferred_element_type=jnp.float32)
    a1 = _mobius_add_bias(_expmap0(mm1, K, sqrtK), ub_lin, K, sqrtK)
    npar = np_ref[...]
    t_a2 = _tclip(npar * _logmap0(a1, K, sqrtK), sqrtK)
    a1_ref[...] = a1
    a2_ref[...] = t_a2


def _tc2_body(c_ref, g2w_ref, g2b_ref, agg_ref, np_ref, a2_ref, h2_ref):
    c = c_ref[0, 0]
    K = 1.0 / c
    sqrtK = jnp.sqrt(K)
    ag = agg_ref[...]
    agg = ag[0] + ag[1]
    npar = np_ref[...]
    u2 = _tclip(_tclip((1.0 - npar) * _tclip(agg, sqrtK), sqrtK)
                + a2_ref[...], sqrtK)
    ub_g2 = _bias_tangent(g2b_ref[...], K, sqrtK)
    mm = jnp.dot(u2, g2w_ref[...], preferred_element_type=jnp.float32)
    h2 = _logmap0(_mobius_add_bias(_expmap0(mm, K, sqrtK), ub_g2, K, sqrtK),
                  K, sqrtK)
    h2_ref[...] = h2


def _tc3_body(c_ref, agg_ref, a1_ref, out_ref):
    c = c_ref[0, 0]
    K = 1.0 / c
    sqrtK = jnp.sqrt(K)
    ag = agg_ref[...]
    agg = ag[0] + ag[1]
    cat = jnp.concatenate([_tclip(agg, sqrtK), a1_ref[...]], axis=-1)
    out_ref[...] = _expmap0(cat, K, sqrtK)


_smem_spec = pl.BlockSpec(memory_space=pltpu.SMEM)


def _full_spec(shape):
    nd = len(shape)
    return pl.BlockSpec(shape, lambda i, _n=nd: (0,) * _n)


def _row_spec(d):
    return pl.BlockSpec((B, d), lambda i: (i, 0))


_tc1h = pl.pallas_call(
    _tc1h_body,
    grid=(GRID,),
    in_specs=[_smem_spec, _full_spec((D, D)), _full_spec((1, D)),
              _row_spec(D)],
    out_specs=[_row_spec(D)],
    out_shape=[jax.ShapeDtypeStruct((N, D), jnp.float32)],
)

_tc1a = pl.pallas_call(
    _tc1a_body,
    grid=(GRID,),
    in_specs=[_smem_spec, _full_spec((D, D)), _full_spec((1, D)),
              _row_spec(D), _row_spec(1)],
    out_specs=[_row_spec(D), _row_spec(D)],
    out_shape=[jax.ShapeDtypeStruct((N, D), jnp.float32)] * 2,
)

_tc2 = pl.pallas_call(
    _tc2_body,
    grid=(GRID,),
    in_specs=[_smem_spec, _full_spec((D, D)), _full_spec((1, D)),
              pl.BlockSpec((2, B, D), lambda i: (0, i, 0)),
              _row_spec(1), _row_spec(D)],
    out_specs=[_row_spec(D)],
    out_shape=[jax.ShapeDtypeStruct((N, D), jnp.float32)],
)

_tc3 = pl.pallas_call(
    _tc3_body,
    grid=(GRID,),
    in_specs=[_smem_spec,
              pl.BlockSpec((2, B, D), lambda i: (0, i, 0)),
              _row_spec(D)],
    out_specs=[_row_spec(2 * D)],
    out_shape=[jax.ShapeDtypeStruct((N, 2 * D), jnp.float32)],
)


def _sc_agg_body(h_hbm, rows_hbm, cols_hbm, adj_hbm, out_hbm,
                 colv, rowv, adjv, rbuf0, rbuf1, rbuf2, aggsh,
                 gsem0, gsem1, gsem2, ssem0, ssem1, ssem2):
    cid = lax.axis_index("c")
    sid = lax.axis_index("s")
    wid = cid * NS + sid

    def zrow(i, carry):
        for j in range(D // 16):
            rbuf0[i, pl.ds(16 * j, 16)] = jnp.zeros((16,), jnp.float32)
        return carry

    lax.fori_loop(0, K_CH, zrow, 0)
    for t in range(STRIPE // K_CH):
        pltpu.sync_copy(rbuf0,
                        aggsh.at[pl.ds(sid * STRIPE + t * K_CH, K_CH)])
    pltpu.sync_copy(rbuf0.at[pl.ds(0, STRIPE % K_CH)],
                    aggsh.at[pl.ds(sid * STRIPE + STRIPE - STRIPE % K_CH,
                                   STRIPE % K_CH)])

    @pl.when(sid == NS - 1)
    def _zero_rem():
        pltpu.sync_copy(rbuf0.at[pl.ds(0, REM)],
                        aggsh.at[pl.ds(NS * STRIPE, REM)])

    plsc.subcore_barrier()

    def _mul(g, rb):
        def mul_blk(eb, c2):
            av = adjv[g, pl.ds(16 * eb, 16)]
            for l in range(16):
                vb = jnp.full((16,), av[l], jnp.float32)
                e = 16 * eb + l
                for j in range(D // 16):
                    rb[e, pl.ds(16 * j, 16)] = rb[e, pl.ds(16 * j, 16)] * vb
            return c2

        lax.fori_loop(0, K_CH // 16, mul_blk, 0)

    def _gissue(g, rb, sem):
        pltpu.async_copy(h_hbm.at[colv.at[g]], rb, sem)

    def _gwait(g, rb, sem):
        pltpu.make_async_copy(h_hbm.at[colv.at[g]], rb, sem).wait()

    def _sissue(g, rb, sem):
        pltpu.async_copy(rb, aggsh.at[rowv.at[g]], sem, add=True)

    def _swait(rb, sem):
        pltpu.make_async_copy(rb, aggsh.at[rowv.at[0]], sem).wait()

    def stage(st, carry):
        pltpu.sync_copy(cols_hbm.at[wid, st], colv)
        pltpu.sync_copy(rows_hbm.at[wid, st], rowv)
        pltpu.sync_copy(adj_hbm.at[wid, st], adjv)

        pltpu.async_copy(h_hbm.at[colv.at[0]], rbuf0, gsem0)
        pltpu.async_copy(h_hbm.at[colv.at[1]], rbuf1, gsem1)

        _gwait(0, rbuf0, gsem0)
        _mul(0, rbuf0)
        _gissue(2, rbuf2, gsem2)
        _sissue(0, rbuf0, ssem0)

        _gwait(1, rbuf1, gsem1)
        _mul(1, rbuf1)
        _swait(rbuf0, ssem0)
        _gissue(3, rbuf0, gsem0)
        _sissue(1, rbuf1, ssem1)

        def core(p, c1):
            k = 3 * p + 2
            _gwait(k, rbuf2, gsem2)
            _mul(k, rbuf2)
            _swait(rbuf1, ssem1)
            _gissue(k + 2, rbuf1, gsem1)
            _sissue(k, rbuf2, ssem2)

            _gwait(k + 1, rbuf0, gsem0)
            _mul(k + 1, rbuf0)
            _swait(rbuf2, ssem2)
            _gissue(k + 3, rbuf2, gsem2)
            _sissue(k + 1, rbuf0, ssem0)

            _gwait(k + 2, rbuf1, gsem1)
            _mul(k + 2, rbuf1)
            _swait(rbuf0, ssem0)
            _gissue(k + 4, rbuf0, gsem0)
            _sissue(k + 2, rbuf1, ssem1)
            return c1

        lax.fori_loop(0, (SB - 4) // 3, core, 0)

        _gwait(SB - 2, rbuf2, gsem2)
        _mul(SB - 2, rbuf2)
        _swait(rbuf1, ssem1)
        _sissue(SB - 2, rbuf2, ssem2)

        _gwait(SB - 1, rbuf0, gsem0)
        _mul(SB - 1, rbuf0)
        _swait(rbuf2, ssem2)
        _sissue(SB - 1, rbuf0, ssem0)

        _swait(rbuf0, ssem0)
        return carry

    lax.fori_loop(0, NST, stage, 0)

    plsc.subcore_barrier()
    pltpu.sync_copy(aggsh.at[pl.ds(sid * STRIPE, STRIPE)],
                    out_hbm.at[cid, pl.ds(sid * STRIPE, STRIPE)])

    @pl.when(sid == NS - 1)
    def _copy_rem():
        pltpu.sync_copy(aggsh.at[pl.ds(NS * STRIPE, REM)],
                        out_hbm.at[cid, pl.ds(NS * STRIPE, REM)])


_sc_agg = functools.partial(
    pl.kernel,
    out_type=jax.ShapeDtypeStruct((NC, N, D), jnp.float32),
    mesh=plsc.VectorSubcoreMesh(core_axis_name="c", subcore_axis_name="s"),
    scratch_types=[
        pltpu.VMEM((SB, K_CH), jnp.int32),
        pltpu.VMEM((SB, K_CH), jnp.int32),
        pltpu.VMEM((SB, K_CH), jnp.float32),
        pltpu.VMEM((K_CH, D), jnp.float32),
        pltpu.VMEM((K_CH, D), jnp.float32),
        pltpu.VMEM((K_CH, D), jnp.float32),
        pltpu.VMEM_SHARED((N, D), jnp.float32),
        pltpu.SemaphoreType.DMA,
        pltpu.SemaphoreType.DMA,
        pltpu.SemaphoreType.DMA,
        pltpu.SemaphoreType.DMA,
        pltpu.SemaphoreType.DMA,
        pltpu.SemaphoreType.DMA,
    ],
)(_sc_agg_body)


def kernel(A1_tensor, adj_values, raw_c, Lin1, Lin1_bias, n_param,
           gc1_w, gc1_b, gc2_w, gc2_b, edge_index):
    c = jax.nn.softplus(raw_c)[0] + 1e-05
    c_arr = jnp.reshape(c, (1, 1))
    linb = Lin1_bias.reshape(1, D)
    g1b = gc1_b.reshape(1, D)
    g2b = gc2_b.reshape(1, D)
    rows2 = edge_index[0].reshape(NW, NST, SB, K_CH)
    cols2 = edge_index[1].reshape(NW, NST, SB, K_CH)
    adj2 = adj_values.reshape(NW, NST, SB, K_CH)

    (h1,) = _tc1h(c_arr, gc1_w, g1b, A1_tensor)
    p1 = _sc_agg(h1, rows2, cols2, adj2)
    a1, a2 = _tc1a(c_arr, Lin1, linb, A1_tensor, n_param)
    (h2,) = _tc2(c_arr, gc2_w, g2b, p1, n_param, a2)
    p2 = _sc_agg(h2, rows2, cols2, adj2)
    (out,) = _tc3(c_arr, p2, a1)
    return out, c


# speedup vs baseline: 2.9322x; 1.0336x over previous
import functools

import jax
import jax.numpy as jnp
from jax import lax
from jax.experimental import pallas as pl
from jax.experimental.pallas import tpu as pltpu
from jax.experimental.pallas import tpu_sc as plsc

N = 10000
E = 320000
D = 128
EPS = 1e-7
MIN_NORM = 1e-15
MAX_NORM = 1e6

NC = 2
NS = 16
NW = NC * NS
EP = E // NW
K_CH = 80
NCH = EP // K_CH
SB = 25
NST = NCH // SB
STRIPE = 624
REM = N - NS * STRIPE

B = 1000
GRID = N // B


def _m0(d):
    return (lax.broadcasted_iota(jnp.int32, (1, d), 1) == 0).astype(jnp.float32)


def _cosh(x):
    e = jnp.exp(jnp.clip(x, -15.0, 15.0))
    return 0.5 * (e + 1.0 / e)


def _sinh(x):
    e = jnp.exp(jnp.clip(x, -15.0, 15.0))
    return 0.5 * (e - 1.0 / e)


def _proj(x, K):
    m0 = _m0(x.shape[-1])
    ysq = jnp.sum(x * x * (1.0 - m0), -1, keepdims=True)
    first = jnp.sqrt(jnp.clip(K + ysq, EPS, None))
    return jnp.where(m0 > 0, first, x)


def _expmap0(u, K, sqrtK):
    m0 = _m0(u.shape[-1])
    xs = u * (1.0 - m0)
    xsq = jnp.sum(xs * xs, -1, keepdims=True)
    x_norm = jnp.sqrt(jnp.clip(xsq, MIN_NORM, None))
    theta = x_norm / sqrtK
    first = sqrtK * _cosh(theta)
    rest = sqrtK * _sinh(theta) * xs / x_norm
    return _proj(jnp.where(m0 > 0, first, rest), K)


def _logmap0(x, K, sqrtK):
    m0 = _m0(x.shape[-1])
    ys = x * (1.0 - m0)
    y_norm = jnp.sqrt(jnp.clip(jnp.sum(ys * ys, -1, keepdims=True), MIN_NORM, None))
    x0 = jnp.sum(x * m0, -1, keepdims=True)
    th = jnp.clip(x0 / sqrtK, 1.0 + EPS, None)
    arc = jnp.log(th + jnp.sqrt(jnp.clip(th * th - 1.0, MIN_NORM, None)))
    return sqrtK * arc * ys / y_norm


def _mobius_add_bias(x, u_b, K, sqrtK):
    m0 = _m0(x.shape[-1])
    x0 = jnp.sum(x * m0, -1, keepdims=True)
    ys = x * (1.0 - m0)
    y_norm = jnp.sqrt(jnp.clip(jnp.sum(ys * ys, -1, keepdims=True), MIN_NORM, None))
    y_unit = ys / y_norm
    v = jnp.where(m0 > 0, -y_norm, (sqrtK - x0) * y_unit)
    alpha = jnp.sum(y_unit * u_b, -1, keepdims=True) / sqrtK
    w = u_b - alpha * v
    ux = jnp.sum(ys * w, -1, keepdims=True)
    first = ux / jnp.clip(x0, MIN_NORM, None)
    v2 = jnp.where(m0 > 0, first, w)
    mdot = jnp.sum(v2 * v2, -1, keepdims=True) - 2.0 * first * first
    normu = jnp.clip(jnp.sqrt(jnp.clip(mdot, EPS, None)), None, MAX_NORM)
    theta = jnp.clip(normu / sqrtK, MIN_NORM, None)
    res = _cosh(theta) * x + _sinh(theta) * v2 / theta
    return _proj(res, K)


def _bias_tangent(b_row, K, sqrtK):
    bias1 = b_row * (1.0 - _m0(b_row.shape[-1]))
    return _logmap0(_proj(_expmap0(bias1, K, sqrtK), K), K, sqrtK)


def _tclip(u, sqrtK):
    m0 = _m0(u.shape[-1])
    us = u * (1.0 - m0)
    n = jnp.sqrt(jnp.clip(jnp.sum(us * us, -1, keepdims=True), MIN_NORM, None))
    return us * jnp.minimum(1.0, 15.0 * sqrtK / n)


def _tc1h_body(c_ref, g1w_ref, g1b_ref, A1_ref, h1_ref):
    c = c_ref[0, 0]
    K = 1.0 / c
    sqrtK = jnp.sqrt(K)
    ua = _tclip(A1_ref[...], sqrtK)
    ub_g1 = _bias_tangent(g1b_ref[...], K, sqrtK)
    mmg = jnp.dot(ua, g1w_ref[...], preferred_element_type=jnp.float32)
    h1 = _logmap0(_mobius_add_bias(_expmap0(mmg, K, sqrtK), ub_g1, K, sqrtK),
                  K, sqrtK)
    h1_ref[...] = h1


def _tc1a_body(c_ref, lin_ref, linb_ref, A1_ref, np_ref, a1_ref, a2_ref):
    c = c_ref[0, 0]
    K = 1.0 / c
    sqrtK = jnp.sqrt(K)
    ua = _tclip(A1_ref[...], sqrtK)
    ub_lin = _bias_tangent(linb_ref[...], K, sqrtK)
    mm1 = jnp.dot(ua, lin_ref[...], preferred_element_type=jnp.float32)
    a1 = _mobius_add_bias(_expmap0(mm1, K, sqrtK), ub_lin, K, sqrtK)
    npar = np_ref[...]
    t_a2 = _tclip(npar * _logmap0(a1, K, sqrtK), sqrtK)
    a1_ref[...] = a1
    a2_ref[...] = t_a2


def _tc2_body(c_ref, g2w_ref, g2b_ref, agg_ref, np_ref, a2_ref, h2_ref):
    c = c_ref[0, 0]
    K = 1.0 / c
    sqrtK = jnp.sqrt(K)
    ag = agg_ref[...]
    agg = ag[0] + ag[1]
    npar = np_ref[...]
    u2 = _tclip(_tclip((1.0 - npar) * _tclip(agg, sqrtK), sqrtK)
                + a2_ref[...], sqrtK)
    ub_g2 = _bias_tangent(g2b_ref[...], K, sqrtK)
    mm = jnp.dot(u2, g2w_ref[...], preferred_element_type=jnp.float32)
    h2 = _logmap0(_mobius_add_bias(_expmap0(mm, K, sqrtK), ub_g2, K, sqrtK),
                  K, sqrtK)
    h2_ref[...] = h2


def _tc3_body(c_ref, agg_ref, a1_ref, out_ref):
    c = c_ref[0, 0]
    K = 1.0 / c
    sqrtK = jnp.sqrt(K)
    ag = agg_ref[...]
    agg = ag[0] + ag[1]
    cat = jnp.concatenate([_tclip(agg, sqrtK), a1_ref[...]], axis=-1)
    out_ref[...] = _expmap0(cat, K, sqrtK)


_smem_spec = pl.BlockSpec(memory_space=pltpu.SMEM)


def _full_spec(shape):
    nd = len(shape)
    return pl.BlockSpec(shape, lambda i, _n=nd: (0,) * _n)


def _row_spec(d):
    return pl.BlockSpec((B, d), lambda i: (i, 0))


_tc1h = pl.pallas_call(
    _tc1h_body,
    grid=(GRID,),
    in_specs=[_smem_spec, _full_spec((D, D)), _full_spec((1, D)),
              _row_spec(D)],
    out_specs=[_row_spec(D)],
    out_shape=[jax.ShapeDtypeStruct((N, D), jnp.float32)],
)

_tc1a = pl.pallas_call(
    _tc1a_body,
    grid=(GRID,),
    in_specs=[_smem_spec, _full_spec((D, D)), _full_spec((1, D)),
              _row_spec(D), _row_spec(1)],
    out_specs=[_row_spec(D), _row_spec(D)],
    out_shape=[jax.ShapeDtypeStruct((N, D), jnp.float32)] * 2,
)

_tc2 = pl.pallas_call(
    _tc2_body,
    grid=(GRID,),
    in_specs=[_smem_spec, _full_spec((D, D)), _full_spec((1, D)),
              pl.BlockSpec((2, B, D), lambda i: (0, i, 0)),
              _row_spec(1), _row_spec(D)],
    out_specs=[_row_spec(D)],
    out_shape=[jax.ShapeDtypeStruct((N, D), jnp.float32)],
)

_tc3 = pl.pallas_call(
    _tc3_body,
    grid=(GRID,),
    in_specs=[_smem_spec,
              pl.BlockSpec((2, B, D), lambda i: (0, i, 0)),
              _row_spec(D)],
    out_specs=[_row_spec(2 * D)],
    out_shape=[jax.ShapeDtypeStruct((N, 2 * D), jnp.float32)],
)


def _sc_agg_body(h_hbm, rows_hbm, cols_hbm, adj_hbm, out_hbm,
                 colv, rowv, adjv, rbuf0, rbuf1, rbuf2, aggsh,
                 gsem0, gsem1, gsem2, ssem0, ssem1, ssem2, isem):
    cid = lax.axis_index("c")
    sid = lax.axis_index("s")
    wid = cid * NS + sid

    def zrow(i, carry):
        for j in range(D // 16):
            rbuf0[i, pl.ds(16 * j, 16)] = jnp.zeros((16,), jnp.float32)
        return carry

    lax.fori_loop(0, K_CH, zrow, 0)
    for t in range(STRIPE // K_CH):
        pltpu.async_copy(rbuf0,
                         aggsh.at[pl.ds(sid * STRIPE + t * K_CH, K_CH)],
                         gsem0)
    pltpu.async_copy(rbuf0.at[pl.ds(0, STRIPE % K_CH)],
                     aggsh.at[pl.ds(sid * STRIPE + STRIPE - STRIPE % K_CH,
                                    STRIPE % K_CH)], gsem0)

    @pl.when(sid == NS - 1)
    def _zero_rem():
        pltpu.sync_copy(rbuf0.at[pl.ds(0, REM)],
                        aggsh.at[pl.ds(NS * STRIPE, REM)])

    for t in range(STRIPE // K_CH):
        pltpu.make_async_copy(
            rbuf0, aggsh.at[pl.ds(sid * STRIPE + t * K_CH, K_CH)],
            gsem0).wait()
    pltpu.make_async_copy(
        rbuf0.at[pl.ds(0, STRIPE % K_CH)],
        aggsh.at[pl.ds(sid * STRIPE + STRIPE - STRIPE % K_CH,
                       STRIPE % K_CH)], gsem0).wait()

    plsc.subcore_barrier()

    def _mul(g, rb):
        def mul_blk(eb, c2):
            av = adjv[g, pl.ds(16 * eb, 16)]
            for l in range(16):
                vb = jnp.full((16,), av[l], jnp.float32)
                e = 16 * eb + l
                for j in range(D // 16):
                    rb[e, pl.ds(16 * j, 16)] = rb[e, pl.ds(16 * j, 16)] * vb
            return c2

        lax.fori_loop(0, K_CH // 16, mul_blk, 0)

    def _gissue(g, rb, sem):
        pltpu.async_copy(h_hbm.at[colv.at[g]], rb, sem)

    def _gwait(g, rb, sem):
        pltpu.make_async_copy(h_hbm.at[colv.at[g]], rb, sem).wait()

    def _sissue(g, rb, sem):
        pltpu.async_copy(rb, aggsh.at[rowv.at[g]], sem, add=True)

    def _swait(rb, sem):
        pltpu.make_async_copy(rb, aggsh.at[rowv.at[0]], sem).wait()

    def stage(st, carry):
        pltpu.async_copy(cols_hbm.at[wid, st], colv, isem)
        pltpu.async_copy(rows_hbm.at[wid, st], rowv, isem)
        pltpu.async_copy(adj_hbm.at[wid, st], adjv, isem)
        pltpu.make_async_copy(cols_hbm.at[wid, st], colv, isem).wait()
        pltpu.make_async_copy(rows_hbm.at[wid, st], rowv, isem).wait()
        pltpu.make_async_copy(adj_hbm.at[wid, st], adjv, isem).wait()

        pltpu.async_copy(h_hbm.at[colv.at[0]], rbuf0, gsem0)
        pltpu.async_copy(h_hbm.at[colv.at[1]], rbuf1, gsem1)

        _gwait(0, rbuf0, gsem0)
        _mul(0, rbuf0)
        _gissue(2, rbuf2, gsem2)
        _sissue(0, rbuf0, ssem0)

        _gwait(1, rbuf1, gsem1)
        _mul(1, rbuf1)
        _swait(rbuf0, ssem0)
        _gissue(3, rbuf0, gsem0)
        _sissue(1, rbuf1, ssem1)

        def core(p, c1):
            k = 3 * p + 2
            _gwait(k, rbuf2, gsem2)
            _mul(k, rbuf2)
            _swait(rbuf1, ssem1)
            _gissue(k + 2, rbuf1, gsem1)
            _sissue(k, rbuf2, ssem2)

            _gwait(k + 1, rbuf0, gsem0)
            _mul(k + 1, rbuf0)
            _swait(rbuf2, ssem2)
            _gissue(k + 3, rbuf2, gsem2)
            _sissue(k + 1, rbuf0, ssem0)

            _gwait(k + 2, rbuf1, gsem1)
            _mul(k + 2, rbuf1)
            _swait(rbuf0, ssem0)
            _gissue(k + 4, rbuf0, gsem0)
            _sissue(k + 2, rbuf1, ssem1)
            return c1

        lax.fori_loop(0, (SB - 4) // 3, core, 0)

        _gwait(SB - 2, rbuf2, gsem2)
        _mul(SB - 2, rbuf2)
        _swait(rbuf1, ssem1)
        _sissue(SB - 2, rbuf2, ssem2)

        _gwait(SB - 1, rbuf0, gsem0)
        _mul(SB - 1, rbuf0)
        _swait(rbuf2, ssem2)
        _sissue(SB - 1, rbuf0, ssem0)

        _swait(rbuf0, ssem0)
        return carry

    lax.fori_loop(0, NST, stage, 0)

    plsc.subcore_barrier()
    pltpu.sync_copy(aggsh.at[pl.ds(sid * STRIPE, STRIPE)],
                    out_hbm.at[cid, pl.ds(sid * STRIPE, STRIPE)])

    @pl.when(sid == NS - 1)
    def _copy_rem():
        pltpu.sync_copy(aggsh.at[pl.ds(NS * STRIPE, REM)],
                        out_hbm.at[cid, pl.ds(NS * STRIPE, REM)])


_sc_agg = functools.partial(
    pl.kernel,
    out_type=jax.ShapeDtypeStruct((NC, N, D), jnp.float32),
    mesh=plsc.VectorSubcoreMesh(core_axis_name="c", subcore_axis_name="s"),
    scratch_types=[
        pltpu.VMEM((SB, K_CH), jnp.int32),
        pltpu.VMEM((SB, K_CH), jnp.int32),
        pltpu.VMEM((SB, K_CH), jnp.float32),
        pltpu.VMEM((K_CH, D), jnp.float32),
        pltpu.VMEM((K_CH, D), jnp.float32),
        pltpu.VMEM((K_CH, D), jnp.float32),
        pltpu.VMEM_SHARED((N, D), jnp.float32),
        pltpu.SemaphoreType.DMA,
        pltpu.SemaphoreType.DMA,
        pltpu.SemaphoreType.DMA,
        pltpu.SemaphoreType.DMA,
        pltpu.SemaphoreType.DMA,
        pltpu.SemaphoreType.DMA,
        pltpu.SemaphoreType.DMA,
    ],
)(_sc_agg_body)


def kernel(A1_tensor, adj_values, raw_c, Lin1, Lin1_bias, n_param,
           gc1_w, gc1_b, gc2_w, gc2_b, edge_index):
    c = jax.nn.softplus(raw_c)[0] + 1e-05
    c_arr = jnp.reshape(c, (1, 1))
    linb = Lin1_bias.reshape(1, D)
    g1b = gc1_b.reshape(1, D)
    g2b = gc2_b.reshape(1, D)
    rows2 = edge_index[0].reshape(NW, NST, SB, K_CH)
    cols2 = edge_index[1].reshape(NW, NST, SB, K_CH)
    adj2 = adj_values.reshape(NW, NST, SB, K_CH)

    (h1,) = _tc1h(c_arr, gc1_w, g1b, A1_tensor)
    p1 = _sc_agg(h1, rows2, cols2, adj2)
    a1, a2 = _tc1a(c_arr, Lin1, linb, A1_tensor, n_param)
    (h2,) = _tc2(c_arr, gc2_w, g2b, p1, n_param, a2)
    p2 = _sc_agg(h2, rows2, cols2, adj2)
    (out,) = _tc3(c_arr, p2, a1)
    return out, c


# speedup vs baseline: 2.9500x; 1.0061x over previous
import functools

import jax
import jax.numpy as jnp
from jax import lax
from jax.experimental import pallas as pl
from jax.experimental.pallas import tpu as pltpu
from jax.experimental.pallas import tpu_sc as plsc

N = 10000
E = 320000
D = 128
EPS = 1e-7
MIN_NORM = 1e-15
MAX_NORM = 1e6

NC = 2
NS = 16
NW = NC * NS
EP = E // NW
K_CH = 80
NCH = EP // K_CH
SB = 25
NST = NCH // SB
STRIPE = 624
REM = N - NS * STRIPE

B = 2000
GRID = N // B


def _m0(d):
    return (lax.broadcasted_iota(jnp.int32, (1, d), 1) == 0).astype(jnp.float32)


def _cosh(x):
    e = jnp.exp(jnp.clip(x, -15.0, 15.0))
    return 0.5 * (e + 1.0 / e)


def _sinh(x):
    e = jnp.exp(jnp.clip(x, -15.0, 15.0))
    return 0.5 * (e - 1.0 / e)


def _proj(x, K):
    m0 = _m0(x.shape[-1])
    ysq = jnp.sum(x * x * (1.0 - m0), -1, keepdims=True)
    first = jnp.sqrt(jnp.clip(K + ysq, EPS, None))
    return jnp.where(m0 > 0, first, x)


def _expmap0(u, K, sqrtK):
    m0 = _m0(u.shape[-1])
    xs = u * (1.0 - m0)
    xsq = jnp.sum(xs * xs, -1, keepdims=True)
    x_norm = jnp.sqrt(jnp.clip(xsq, MIN_NORM, None))
    theta = x_norm / sqrtK
    first = sqrtK * _cosh(theta)
    rest = sqrtK * _sinh(theta) * xs / x_norm
    return _proj(jnp.where(m0 > 0, first, rest), K)


def _logmap0(x, K, sqrtK):
    m0 = _m0(x.shape[-1])
    ys = x * (1.0 - m0)
    y_norm = jnp.sqrt(jnp.clip(jnp.sum(ys * ys, -1, keepdims=True), MIN_NORM, None))
    x0 = jnp.sum(x * m0, -1, keepdims=True)
    th = jnp.clip(x0 / sqrtK, 1.0 + EPS, None)
    arc = jnp.log(th + jnp.sqrt(jnp.clip(th * th - 1.0, MIN_NORM, None)))
    return sqrtK * arc * ys / y_norm


def _mobius_add_bias(x, u_b, K, sqrtK):
    m0 = _m0(x.shape[-1])
    x0 = jnp.sum(x * m0, -1, keepdims=True)
    ys = x * (1.0 - m0)
    y_norm = jnp.sqrt(jnp.clip(jnp.sum(ys * ys, -1, keepdims=True), MIN_NORM, None))
    y_unit = ys / y_norm
    v = jnp.where(m0 > 0, -y_norm, (sqrtK - x0) * y_unit)
    alpha = jnp.sum(y_unit * u_b, -1, keepdims=True) / sqrtK
    w = u_b - alpha * v
    ux = jnp.sum(ys * w, -1, keepdims=True)
    first = ux / jnp.clip(x0, MIN_NORM, None)
    v2 = jnp.where(m0 > 0, first, w)
    mdot = jnp.sum(v2 * v2, -1, keepdims=True) - 2.0 * first * first
    normu = jnp.clip(jnp.sqrt(jnp.clip(mdot, EPS, None)), None, MAX_NORM)
    theta = jnp.clip(normu / sqrtK, MIN_NORM, None)
    res = _cosh(theta) * x + _sinh(theta) * v2 / theta
    return _proj(res, K)


def _bias_tangent(b_row, K, sqrtK):
    bias1 = b_row * (1.0 - _m0(b_row.shape[-1]))
    return _logmap0(_proj(_expmap0(bias1, K, sqrtK), K), K, sqrtK)


def _tclip(u, sqrtK):
    m0 = _m0(u.shape[-1])
    us = u * (1.0 - m0)
    n = jnp.sqrt(jnp.clip(jnp.sum(us * us, -1, keepdims=True), MIN_NORM, None))
    return us * jnp.minimum(1.0, 15.0 * sqrtK / n)


def _tc1h_body(c_ref, g1w_ref, g1b_ref, A1_ref, h1_ref):
    c = c_ref[0, 0]
    K = 1.0 / c
    sqrtK = jnp.sqrt(K)
    ua = _tclip(A1_ref[...], sqrtK)
    ub_g1 = _bias_tangent(g1b_ref[...], K, sqrtK)
    mmg = jnp.dot(ua, g1w_ref[...], preferred_element_type=jnp.float32)
    h1 = _logmap0(_mobius_add_bias(_expmap0(mmg, K, sqrtK), ub_g1, K, sqrtK),
                  K, sqrtK)
    h1_ref[...] = h1


def _tc1a_body(c_ref, lin_ref, linb_ref, A1_ref, np_ref, a1_ref, a2_ref):
    c = c_ref[0, 0]
    K = 1.0 / c
    sqrtK = jnp.sqrt(K)
    ua = _tclip(A1_ref[...], sqrtK)
    ub_lin = _bias_tangent(linb_ref[...], K, sqrtK)
    mm1 = jnp.dot(ua, lin_ref[...], preferred_element_type=jnp.float32)
    a1 = _mobius_add_bias(_expmap0(mm1, K, sqrtK), ub_lin, K, sqrtK)
    npar = np_ref[...]
    t_a2 = _tclip(npar * _logmap0(a1, K, sqrtK), sqrtK)
    a1_ref[...] = a1
    a2_ref[...] = t_a2


def _tc2_body(c_ref, g2w_ref, g2b_ref, agg_ref, np_ref, a2_ref, h2_ref):
    c = c_ref[0, 0]
    K = 1.0 / c
    sqrtK = jnp.sqrt(K)
    ag = agg_ref[...]
    agg = ag[0] + ag[1]
    npar = np_ref[...]
    u2 = _tclip(_tclip((1.0 - npar) * _tclip(agg, sqrtK), sqrtK)
                + a2_ref[...], sqrtK)
    ub_g2 = _bias_tangent(g2b_ref[...], K, sqrtK)
    mm = jnp.dot(u2, g2w_ref[...], preferred_element_type=jnp.float32)
    h2 = _logmap0(_mobius_add_bias(_expmap0(mm, K, sqrtK), ub_g2, K, sqrtK),
                  K, sqrtK)
    h2_ref[...] = h2


def _tc3_body(c_ref, agg_ref, a1_ref, out_ref):
    c = c_ref[0, 0]
    K = 1.0 / c
    sqrtK = jnp.sqrt(K)
    ag = agg_ref[...]
    agg = ag[0] + ag[1]
    cat = jnp.concatenate([_tclip(agg, sqrtK), a1_ref[...]], axis=-1)
    out_ref[...] = _expmap0(cat, K, sqrtK)


_smem_spec = pl.BlockSpec(memory_space=pltpu.SMEM)


def _full_spec(shape):
    nd = len(shape)
    return pl.BlockSpec(shape, lambda i, _n=nd: (0,) * _n)


def _row_spec(d):
    return pl.BlockSpec((B, d), lambda i: (i, 0))


_tc1h = pl.pallas_call(
    _tc1h_body,
    grid=(GRID,),
    in_specs=[_smem_spec, _full_spec((D, D)), _full_spec((1, D)),
              _row_spec(D)],
    out_specs=[_row_spec(D)],
    out_shape=[jax.ShapeDtypeStruct((N, D), jnp.float32)],
)

_tc1a = pl.pallas_call(
    _tc1a_body,
    grid=(GRID,),
    in_specs=[_smem_spec, _full_spec((D, D)), _full_spec((1, D)),
              _row_spec(D), _row_spec(1)],
    out_specs=[_row_spec(D), _row_spec(D)],
    out_shape=[jax.ShapeDtypeStruct((N, D), jnp.float32)] * 2,
)

_tc2 = pl.pallas_call(
    _tc2_body,
    grid=(GRID,),
    in_specs=[_smem_spec, _full_spec((D, D)), _full_spec((1, D)),
              pl.BlockSpec((2, B, D), lambda i: (0, i, 0)),
              _row_spec(1), _row_spec(D)],
    out_specs=[_row_spec(D)],
    out_shape=[jax.ShapeDtypeStruct((N, D), jnp.float32)],
)

_tc3 = pl.pallas_call(
    _tc3_body,
    grid=(GRID,),
    in_specs=[_smem_spec,
              pl.BlockSpec((2, B, D), lambda i: (0, i, 0)),
              _row_spec(D)],
    out_specs=[_row_spec(2 * D)],
    out_shape=[jax.ShapeDtypeStruct((N, 2 * D), jnp.float32)],
)


def _sc_agg_body(h_hbm, rows_hbm, cols_hbm, adj_hbm, out_hbm,
                 colv, rowv, adjv, rbuf0, rbuf1, rbuf2, aggsh,
                 gsem0, gsem1, gsem2, ssem0, ssem1, ssem2, isem):
    cid = lax.axis_index("c")
    sid = lax.axis_index("s")
    wid = cid * NS + sid

    def zrow(i, carry):
        for j in range(D // 16):
            rbuf0[i, pl.ds(16 * j, 16)] = jnp.zeros((16,), jnp.float32)
        return carry

    lax.fori_loop(0, K_CH, zrow, 0)
    for t in range(STRIPE // K_CH):
        pltpu.async_copy(rbuf0,
                         aggsh.at[pl.ds(sid * STRIPE + t * K_CH, K_CH)],
                         gsem0)
    pltpu.async_copy(rbuf0.at[pl.ds(0, STRIPE % K_CH)],
                     aggsh.at[pl.ds(sid * STRIPE + STRIPE - STRIPE % K_CH,
                                    STRIPE % K_CH)], gsem0)

    @pl.when(sid == NS - 1)
    def _zero_rem():
        pltpu.sync_copy(rbuf0.at[pl.ds(0, REM)],
                        aggsh.at[pl.ds(NS * STRIPE, REM)])

    for t in range(STRIPE // K_CH):
        pltpu.make_async_copy(
            rbuf0, aggsh.at[pl.ds(sid * STRIPE + t * K_CH, K_CH)],
            gsem0).wait()
    pltpu.make_async_copy(
        rbuf0.at[pl.ds(0, STRIPE % K_CH)],
        aggsh.at[pl.ds(sid * STRIPE + STRIPE - STRIPE % K_CH,
                       STRIPE % K_CH)], gsem0).wait()

    plsc.subcore_barrier()

    def _mul(g, rb):
        def mul_blk(eb, c2):
            av = adjv[g, pl.ds(16 * eb, 16)]
            for l in range(16):
                vb = jnp.full((16,), av[l], jnp.float32)
                e = 16 * eb + l
                for j in range(D // 16):
                    rb[e, pl.ds(16 * j, 16)] = rb[e, pl.ds(16 * j, 16)] * vb
            return c2

        lax.fori_loop(0, K_CH // 16, mul_blk, 0)

    def _gissue(g, rb, sem):
        pltpu.async_copy(h_hbm.at[colv.at[g]], rb, sem)

    def _gwait(g, rb, sem):
        pltpu.make_async_copy(h_hbm.at[colv.at[g]], rb, sem).wait()

    def _sissue(g, rb, sem):
        pltpu.async_copy(rb, aggsh.at[rowv.at[g]], sem, add=True)

    def _swait(rb, sem):
        pltpu.make_async_copy(rb, aggsh.at[rowv.at[0]], sem).wait()

    def stage(st, carry):
        pltpu.async_copy(cols_hbm.at[wid, st], colv, isem)
        pltpu.async_copy(rows_hbm.at[wid, st], rowv, isem)
        pltpu.async_copy(adj_hbm.at[wid, st], adjv, isem)
        pltpu.make_async_copy(cols_hbm.at[wid, st], colv, isem).wait()
        pltpu.make_async_copy(rows_hbm.at[wid, st], rowv, isem).wait()
        pltpu.make_async_copy(adj_hbm.at[wid, st], adjv, isem).wait()

        pltpu.async_copy(h_hbm.at[colv.at[0]], rbuf0, gsem0)
        pltpu.async_copy(h_hbm.at[colv.at[1]], rbuf1, gsem1)

        _gwait(0, rbuf0, gsem0)
        _mul(0, rbuf0)
        _gissue(2, rbuf2, gsem2)
        _sissue(0, rbuf0, ssem0)

        _gwait(1, rbuf1, gsem1)
        _mul(1, rbuf1)
        _swait(rbuf0, ssem0)
        _gissue(3, rbuf0, gsem0)
        _sissue(1, rbuf1, ssem1)

        def core(p, c1):
            k = 3 * p + 2
            _gwait(k, rbuf2, gsem2)
            _mul(k, rbuf2)
            _swait(rbuf1, ssem1)
            _gissue(k + 2, rbuf1, gsem1)
            _sissue(k, rbuf2, ssem2)

            _gwait(k + 1, rbuf0, gsem0)
            _mul(k + 1, rbuf0)
            _swait(rbuf2, ssem2)
            _gissue(k + 3, rbuf2, gsem2)
            _sissue(k + 1, rbuf0, ssem0)

            _gwait(k + 2, rbuf1, gsem1)
            _mul(k + 2, rbuf1)
            _swait(rbuf0, ssem0)
            _gissue(k + 4, rbuf0, gsem0)
            _sissue(k + 2, rbuf1, ssem1)
            return c1

        lax.fori_loop(0, (SB - 4) // 3, core, 0)

        _gwait(SB - 2, rbuf2, gsem2)
        _mul(SB - 2, rbuf2)
        _swait(rbuf1, ssem1)
        _sissue(SB - 2, rbuf2, ssem2)

        _gwait(SB - 1, rbuf0, gsem0)
        _mul(SB - 1, rbuf0)
        _swait(rbuf2, ssem2)
        _sissue(SB - 1, rbuf0, ssem0)

        _swait(rbuf0, ssem0)
        return carry

    lax.fori_loop(0, NST, stage, 0)

    plsc.subcore_barrier()
    pltpu.sync_copy(aggsh.at[pl.ds(sid * STRIPE, STRIPE)],
                    out_hbm.at[cid, pl.ds(sid * STRIPE, STRIPE)])

    @pl.when(sid == NS - 1)
    def _copy_rem():
        pltpu.sync_copy(aggsh.at[pl.ds(NS * STRIPE, REM)],
                        out_hbm.at[cid, pl.ds(NS * STRIPE, REM)])


_sc_agg = functools.partial(
    pl.kernel,
    out_type=jax.ShapeDtypeStruct((NC, N, D), jnp.float32),
    mesh=plsc.VectorSubcoreMesh(core_axis_name="c", subcore_axis_name="s"),
    scratch_types=[
        pltpu.VMEM((SB, K_CH), jnp.int32),
        pltpu.VMEM((SB, K_CH), jnp.int32),
        pltpu.VMEM((SB, K_CH), jnp.float32),
        pltpu.VMEM((K_CH, D), jnp.float32),
        pltpu.VMEM((K_CH, D), jnp.float32),
        pltpu.VMEM((K_CH, D), jnp.float32),
        pltpu.VMEM_SHARED((N, D), jnp.float32),
        pltpu.SemaphoreType.DMA,
        pltpu.SemaphoreType.DMA,
        pltpu.SemaphoreType.DMA,
        pltpu.SemaphoreType.DMA,
        pltpu.SemaphoreType.DMA,
        pltpu.SemaphoreType.DMA,
        pltpu.SemaphoreType.DMA,
    ],
)(_sc_agg_body)


def kernel(A1_tensor, adj_values, raw_c, Lin1, Lin1_bias, n_param,
           gc1_w, gc1_b, gc2_w, gc2_b, edge_index):
    c = jax.nn.softplus(raw_c)[0] + 1e-05
    c_arr = jnp.reshape(c, (1, 1))
    linb = Lin1_bias.reshape(1, D)
    g1b = gc1_b.reshape(1, D)
    g2b = gc2_b.reshape(1, D)
    rows2 = edge_index[0].reshape(NW, NST, SB, K_CH)
    cols2 = edge_index[1].reshape(NW, NST, SB, K_CH)
    adj2 = adj_values.reshape(NW, NST, SB, K_CH)

    (h1,) = _tc1h(c_arr, gc1_w, g1b, A1_tensor)
    p1 = _sc_agg(h1, rows2, cols2, adj2)
    a1, a2 = _tc1a(c_arr, Lin1, linb, A1_tensor, n_param)
    (h2,) = _tc2(c_arr, gc2_w, g2b, p1, n_param, a2)
    p2 = _sc_agg(h2, rows2, cols2, adj2)
    (out,) = _tc3(c_arr, p2, a1)
    return out, c


# speedup vs baseline: 2.9980x; 1.0163x over previous
import functools

import jax
import jax.numpy as jnp
from jax import lax
from jax.experimental import pallas as pl
from jax.experimental.pallas import tpu as pltpu
from jax.experimental.pallas import tpu_sc as plsc

N = 10000
E = 320000
D = 128
EPS = 1e-7
MIN_NORM = 1e-15
MAX_NORM = 1e6

NC = 2
NS = 16
NW = NC * NS
EP = E // NW
K_CH = 80
NCH = EP // K_CH
SB = 25
NST = NCH // SB
STRIPE = 624
REM = N - NS * STRIPE

B = 2000
GRID = N // B


def _m0(d):
    return (lax.broadcasted_iota(jnp.int32, (1, d), 1) == 0).astype(jnp.float32)


def _cosh(x):
    e = jnp.exp(jnp.clip(x, -15.0, 15.0))
    return 0.5 * (e + 1.0 / e)


def _sinh(x):
    e = jnp.exp(jnp.clip(x, -15.0, 15.0))
    return 0.5 * (e - 1.0 / e)


def _proj(x, K):
    m0 = _m0(x.shape[-1])
    ysq = jnp.sum(x * x * (1.0 - m0), -1, keepdims=True)
    first = jnp.sqrt(jnp.clip(K + ysq, EPS, None))
    return jnp.where(m0 > 0, first, x)


def _expmap0(u, K, sqrtK):
    m0 = _m0(u.shape[-1])
    xs = u * (1.0 - m0)
    xsq = jnp.sum(xs * xs, -1, keepdims=True)
    x_norm = jnp.sqrt(jnp.clip(xsq, MIN_NORM, None))
    theta = x_norm / sqrtK
    first = sqrtK * _cosh(theta)
    rest = sqrtK * _sinh(theta) * xs / x_norm
    return _proj(jnp.where(m0 > 0, first, rest), K)


def _logmap0(x, K, sqrtK):
    m0 = _m0(x.shape[-1])
    ys = x * (1.0 - m0)
    y_norm = jnp.sqrt(jnp.clip(jnp.sum(ys * ys, -1, keepdims=True), MIN_NORM, None))
    x0 = jnp.sum(x * m0, -1, keepdims=True)
    th = jnp.clip(x0 / sqrtK, 1.0 + EPS, None)
    arc = jnp.log(th + jnp.sqrt(jnp.clip(th * th - 1.0, MIN_NORM, None)))
    return sqrtK * arc * ys / y_norm


def _mobius_add_bias(x, u_b, K, sqrtK):
    m0 = _m0(x.shape[-1])
    x0 = jnp.sum(x * m0, -1, keepdims=True)
    ys = x * (1.0 - m0)
    y_norm = jnp.sqrt(jnp.clip(jnp.sum(ys * ys, -1, keepdims=True), MIN_NORM, None))
    y_unit = ys / y_norm
    v = jnp.where(m0 > 0, -y_norm, (sqrtK - x0) * y_unit)
    alpha = jnp.sum(y_unit * u_b, -1, keepdims=True) / sqrtK
    w = u_b - alpha * v
    ux = jnp.sum(ys * w, -1, keepdims=True)
    first = ux / jnp.clip(x0, MIN_NORM, None)
    v2 = jnp.where(m0 > 0, first, w)
    mdot = jnp.sum(v2 * v2, -1, keepdims=True) - 2.0 * first * first
    normu = jnp.clip(jnp.sqrt(jnp.clip(mdot, EPS, None)), None, MAX_NORM)
    theta = jnp.clip(normu / sqrtK, MIN_NORM, None)
    res = _cosh(theta) * x + _sinh(theta) * v2 / theta
    return _proj(res, K)


def _bias_tangent(b_row, K, sqrtK):
    bias1 = b_row * (1.0 - _m0(b_row.shape[-1]))
    return _logmap0(_proj(_expmap0(bias1, K, sqrtK), K), K, sqrtK)


def _gc_tangent(mm, ub, K, sqrtK):
    m0 = _m0(mm.shape[-1])
    im0 = 1.0 - m0
    xs = mm * im0
    xn = jnp.sqrt(jnp.clip(jnp.sum(xs * xs, -1, keepdims=True), MIN_NORM, None))
    th1 = xn / sqrtK
    e1 = jnp.exp(jnp.clip(th1, -15.0, 15.0))
    sh1 = 0.5 * (e1 - 1.0 / e1)
    y = sqrtK * sh1 * xs / xn
    ysq = jnp.sum(y * y, -1, keepdims=True)
    x0 = jnp.sqrt(jnp.clip(K + ysq, EPS, None))
    y_norm = jnp.sqrt(jnp.clip(ysq, MIN_NORM, None))
    y_unit = y / y_norm
    v = jnp.where(m0 > 0, -y_norm, (sqrtK - x0) * y_unit)
    alpha = jnp.sum(y_unit * ub, -1, keepdims=True) / sqrtK
    w = ub - alpha * v
    ux = jnp.sum(y * w, -1, keepdims=True)
    first2 = ux / jnp.clip(x0, MIN_NORM, None)
    v2 = jnp.where(m0 > 0, first2, w)
    mdot = jnp.sum(v2 * v2, -1, keepdims=True) - 2.0 * first2 * first2
    normu = jnp.clip(jnp.sqrt(jnp.clip(mdot, EPS, None)), None, MAX_NORM)
    th2 = jnp.clip(normu / sqrtK, MIN_NORM, None)
    e2 = jnp.exp(jnp.clip(th2, -15.0, 15.0))
    ch2 = 0.5 * (e2 + 1.0 / e2)
    sh2 = 0.5 * (e2 - 1.0 / e2)
    x_pt = jnp.where(m0 > 0, x0, y)
    res_y = (ch2 * x_pt + sh2 * v2 / th2) * im0
    rsq = jnp.sum(res_y * res_y, -1, keepdims=True)
    x0b = jnp.sqrt(jnp.clip(K + rsq, EPS, None))
    yn2 = jnp.sqrt(jnp.clip(rsq, MIN_NORM, None))
    tharg = jnp.clip(x0b / sqrtK, 1.0 + EPS, None)
    arc = jnp.log(tharg + jnp.sqrt(jnp.clip(tharg * tharg - 1.0, MIN_NORM,
                                            None)))
    return sqrtK * arc * res_y / yn2


def _tclip(u, sqrtK):
    m0 = _m0(u.shape[-1])
    us = u * (1.0 - m0)
    n = jnp.sqrt(jnp.clip(jnp.sum(us * us, -1, keepdims=True), MIN_NORM, None))
    return us * jnp.minimum(1.0, 15.0 * sqrtK / n)


def _tc1h_body(c_ref, g1w_ref, g1b_ref, A1_ref, h1_ref):
    c = c_ref[0, 0]
    K = 1.0 / c
    sqrtK = jnp.sqrt(K)
    ua = _tclip(A1_ref[...], sqrtK)
    ub_g1 = _bias_tangent(g1b_ref[...], K, sqrtK)
    mmg = jnp.dot(ua, g1w_ref[...], preferred_element_type=jnp.float32)
    h1_ref[...] = _gc_tangent(mmg, ub_g1, K, sqrtK)


def _tc1a_body(c_ref, lin_ref, linb_ref, A1_ref, np_ref, a1_ref, a2_ref):
    c = c_ref[0, 0]
    K = 1.0 / c
    sqrtK = jnp.sqrt(K)
    ua = _tclip(A1_ref[...], sqrtK)
    ub_lin = _bias_tangent(linb_ref[...], K, sqrtK)
    mm1 = jnp.dot(ua, lin_ref[...], preferred_element_type=jnp.float32)
    a1 = _mobius_add_bias(_expmap0(mm1, K, sqrtK), ub_lin, K, sqrtK)
    npar = np_ref[...]
    t_a2 = _tclip(npar * _logmap0(a1, K, sqrtK), sqrtK)
    a1_ref[...] = a1
    a2_ref[...] = t_a2


def _tc2_body(c_ref, g2w_ref, g2b_ref, agg_ref, np_ref, a2_ref, h2_ref):
    c = c_ref[0, 0]
    K = 1.0 / c
    sqrtK = jnp.sqrt(K)
    ag = agg_ref[...]
    agg = ag[0] + ag[1]
    npar = np_ref[...]
    u2 = _tclip(_tclip((1.0 - npar) * _tclip(agg, sqrtK), sqrtK)
                + a2_ref[...], sqrtK)
    ub_g2 = _bias_tangent(g2b_ref[...], K, sqrtK)
    mm = jnp.dot(u2, g2w_ref[...], preferred_element_type=jnp.float32)
    h2_ref[...] = _gc_tangent(mm, ub_g2, K, sqrtK)


def _tc3_body(c_ref, agg_ref, a1_ref, out_ref):
    c = c_ref[0, 0]
    K = 1.0 / c
    sqrtK = jnp.sqrt(K)
    ag = agg_ref[...]
    agg = ag[0] + ag[1]
    cat = jnp.concatenate([_tclip(agg, sqrtK), a1_ref[...]], axis=-1)
    out_ref[...] = _expmap0(cat, K, sqrtK)


_smem_spec = pl.BlockSpec(memory_space=pltpu.SMEM)


def _full_spec(shape):
    nd = len(shape)
    return pl.BlockSpec(shape, lambda i, _n=nd: (0,) * _n)


def _row_spec(d):
    return pl.BlockSpec((B, d), lambda i: (i, 0))


_tc1h = pl.pallas_call(
    _tc1h_body,
    grid=(GRID,),
    in_specs=[_smem_spec, _full_spec((D, D)), _full_spec((1, D)),
              _row_spec(D)],
    out_specs=[_row_spec(D)],
    out_shape=[jax.ShapeDtypeStruct((N, D), jnp.float32)],
)

_tc1a = pl.pallas_call(
    _tc1a_body,
    grid=(GRID,),
    in_specs=[_smem_spec, _full_spec((D, D)), _full_spec((1, D)),
              _row_spec(D), _row_spec(1)],
    out_specs=[_row_spec(D), _row_spec(D)],
    out_shape=[jax.ShapeDtypeStruct((N, D), jnp.float32)] * 2,
)

_tc2 = pl.pallas_call(
    _tc2_body,
    grid=(GRID,),
    in_specs=[_smem_spec, _full_spec((D, D)), _full_spec((1, D)),
              pl.BlockSpec((2, B, D), lambda i: (0, i, 0)),
              _row_spec(1), _row_spec(D)],
    out_specs=[_row_spec(D)],
    out_shape=[jax.ShapeDtypeStruct((N, D), jnp.float32)],
)

_tc3 = pl.pallas_call(
    _tc3_body,
    grid=(GRID,),
    in_specs=[_smem_spec,
              pl.BlockSpec((2, B, D), lambda i: (0, i, 0)),
              _row_spec(D)],
    out_specs=[_row_spec(2 * D)],
    out_shape=[jax.ShapeDtypeStruct((N, 2 * D), jnp.float32)],
)


def _sc_agg_body(h_hbm, rows_hbm, cols_hbm, adj_hbm, out_hbm,
                 colv, rowv, adjv, rbuf0, rbuf1, rbuf2, aggsh,
                 gsem0, gsem1, gsem2, ssem0, ssem1, ssem2, isem):
    cid = lax.axis_index("c")
    sid = lax.axis_index("s")
    wid = cid * NS + sid

    def zrow(i, carry):
        for j in range(D // 16):
            rbuf0[i, pl.ds(16 * j, 16)] = jnp.zeros((16,), jnp.float32)
        return carry

    lax.fori_loop(0, K_CH, zrow, 0)
    for t in range(STRIPE // K_CH):
        pltpu.async_copy(rbuf0,
                         aggsh.at[pl.ds(sid * STRIPE + t * K_CH, K_CH)],
                         gsem0)
    pltpu.async_copy(rbuf0.at[pl.ds(0, STRIPE % K_CH)],
                     aggsh.at[pl.ds(sid * STRIPE + STRIPE - STRIPE % K_CH,
                                    STRIPE % K_CH)], gsem0)

    @pl.when(sid == NS - 1)
    def _zero_rem():
        pltpu.sync_copy(rbuf0.at[pl.ds(0, REM)],
                        aggsh.at[pl.ds(NS * STRIPE, REM)])

    for t in range(STRIPE // K_CH):
        pltpu.make_async_copy(
            rbuf0, aggsh.at[pl.ds(sid * STRIPE + t * K_CH, K_CH)],
            gsem0).wait()
    pltpu.make_async_copy(
        rbuf0.at[pl.ds(0, STRIPE % K_CH)],
        aggsh.at[pl.ds(sid * STRIPE + STRIPE - STRIPE % K_CH,
                       STRIPE % K_CH)], gsem0).wait()

    plsc.subcore_barrier()

    def _mul(g, rb):
        def mul_blk(eb, c2):
            av = adjv[g, pl.ds(16 * eb, 16)]
            for l in range(16):
                vb = jnp.full((16,), av[l], jnp.float32)
                e = 16 * eb + l
                for j in range(D // 16):
                    rb[e, pl.ds(16 * j, 16)] = rb[e, pl.ds(16 * j, 16)] * vb
            return c2

        lax.fori_loop(0, K_CH // 16, mul_blk, 0)

    def _gissue(g, rb, sem):
        pltpu.async_copy(h_hbm.at[colv.at[g]], rb, sem)

    def _gwait(g, rb, sem):
        pltpu.make_async_copy(h_hbm.at[colv.at[g]], rb, sem).wait()

    def _sissue(g, rb, sem):
        pltpu.async_copy(rb, aggsh.at[rowv.at[g]], sem, add=True)

    def _swait(rb, sem):
        pltpu.make_async_copy(rb, aggsh.at[rowv.at[0]], sem).wait()

    def stage(st, carry):
        pltpu.async_copy(cols_hbm.at[wid, st], colv, isem)
        pltpu.async_copy(rows_hbm.at[wid, st], rowv, isem)
        pltpu.async_copy(adj_hbm.at[wid, st], adjv, isem)
        pltpu.make_async_copy(cols_hbm.at[wid, st], colv, isem).wait()
        pltpu.make_async_copy(rows_hbm.at[wid, st], rowv, isem).wait()
        pltpu.make_async_copy(adj_hbm.at[wid, st], adjv, isem).wait()

        pltpu.async_copy(h_hbm.at[colv.at[0]], rbuf0, gsem0)
        pltpu.async_copy(h_hbm.at[colv.at[1]], rbuf1, gsem1)

        _gwait(0, rbuf0, gsem0)
        _mul(0, rbuf0)
        _gissue(2, rbuf2, gsem2)
        _sissue(0, rbuf0, ssem0)

        _gwait(1, rbuf1, gsem1)
        _mul(1, rbuf1)
        _swait(rbuf0, ssem0)
        _gissue(3, rbuf0, gsem0)
        _sissue(1, rbuf1, ssem1)

        def core(p, c1):
            k = 3 * p + 2
            _gwait(k, rbuf2, gsem2)
            _mul(k, rbuf2)
            _swait(rbuf1, ssem1)
            _gissue(k + 2, rbuf1, gsem1)
            _sissue(k, rbuf2, ssem2)

            _gwait(k + 1, rbuf0, gsem0)
            _mul(k + 1, rbuf0)
            _swait(rbuf2, ssem2)
            _gissue(k + 3, rbuf2, gsem2)
            _sissue(k + 1, rbuf0, ssem0)

            _gwait(k + 2, rbuf1, gsem1)
            _mul(k + 2, rbuf1)
            _swait(rbuf0, ssem0)
            _gissue(k + 4, rbuf0, gsem0)
            _sissue(k + 2, rbuf1, ssem1)
            return c1

        lax.fori_loop(0, (SB - 4) // 3, core, 0)

        _gwait(SB - 2, rbuf2, gsem2)
        _mul(SB - 2, rbuf2)
        _swait(rbuf1, ssem1)
        _sissue(SB - 2, rbuf2, ssem2)

        _gwait(SB - 1, rbuf0, gsem0)
        _mul(SB - 1, rbuf0)
        _swait(rbuf2, ssem2)
        _sissue(SB - 1, rbuf0, ssem0)

        _swait(rbuf0, ssem0)
        return carry

    lax.fori_loop(0, NST, stage, 0)

    plsc.subcore_barrier()
    pltpu.sync_copy(aggsh.at[pl.ds(sid * STRIPE, STRIPE)],
                    out_hbm.at[cid, pl.ds(sid * STRIPE, STRIPE)])

    @pl.when(sid == NS - 1)
    def _copy_rem():
        pltpu.sync_copy(aggsh.at[pl.ds(NS * STRIPE, REM)],
                        out_hbm.at[cid, pl.ds(NS * STRIPE, REM)])


_sc_agg = functools.partial(
    pl.kernel,
    out_type=jax.ShapeDtypeStruct((NC, N, D), jnp.float32),
    mesh=plsc.VectorSubcoreMesh(core_axis_name="c", subcore_axis_name="s"),
    scratch_types=[
        pltpu.VMEM((SB, K_CH), jnp.int32),
        pltpu.VMEM((SB, K_CH), jnp.int32),
        pltpu.VMEM((SB, K_CH), jnp.float32),
        pltpu.VMEM((K_CH, D), jnp.float32),
        pltpu.VMEM((K_CH, D), jnp.float32),
        pltpu.VMEM((K_CH, D), jnp.float32),
        pltpu.VMEM_SHARED((N, D), jnp.float32),
        pltpu.SemaphoreType.DMA,
        pltpu.SemaphoreType.DMA,
        pltpu.SemaphoreType.DMA,
        pltpu.SemaphoreType.DMA,
        pltpu.SemaphoreType.DMA,
        pltpu.SemaphoreType.DMA,
        pltpu.SemaphoreType.DMA,
    ],
)(_sc_agg_body)


def kernel(A1_tensor, adj_values, raw_c, Lin1, Lin1_bias, n_param,
           gc1_w, gc1_b, gc2_w, gc2_b, edge_index):
    c = jax.nn.softplus(raw_c)[0] + 1e-05
    c_arr = jnp.reshape(c, (1, 1))
    linb = Lin1_bias.reshape(1, D)
    g1b = gc1_b.reshape(1, D)
    g2b = gc2_b.reshape(1, D)
    rows2 = edge_index[0].reshape(NW, NST, SB, K_CH)
    cols2 = edge_index[1].reshape(NW, NST, SB, K_CH)
    adj2 = adj_values.reshape(NW, NST, SB, K_CH)

    (h1,) = _tc1h(c_arr, gc1_w, g1b, A1_tensor)
    p1 = _sc_agg(h1, rows2, cols2, adj2)
    a1, a2 = _tc1a(c_arr, Lin1, linb, A1_tensor, n_param)
    (h2,) = _tc2(c_arr, gc2_w, g2b, p1, n_param, a2)
    p2 = _sc_agg(h2, rows2, cols2, adj2)
    (out,) = _tc3(c_arr, p2, a1)
    return out, c
